# baseline (speedup 1.0000x reference)
import jax
import jax.numpy as jnp
from jax.experimental import pallas as pl


def _mlp(x, W1, b1, W2, b2):
    return jnp.tanh(x @ W1 + b1) @ W2 + b2


def _seg(h, e, n):
    return jax.ops.segment_sum(h[e[0]], e[1], num_segments=n)


def _identity_pallas(x):
    def body(x_ref, o_ref):
        o_ref[...] = x_ref[...]
    bn = 8192
    grid = (pl.cdiv(x.shape[0], bn),)
    return pl.pallas_call(
        body,
        grid=grid,
        in_specs=[pl.BlockSpec((bn, x.shape[1]), lambda i: (i, 0))],
        out_specs=pl.BlockSpec((bn, x.shape[1]), lambda i: (i, 0)),
        out_shape=jax.ShapeDtypeStruct(x.shape, x.dtype))(x)


def kernel(h1, h2, h3, h4, Wup2_1, bup2_1, Wup2_2, bup2_2, Wup3_1, bup3_1, Wup3_2, bup3_2, Wup4_1, bup4_1, Wup4_2, bup4_2, Wdn1_1, bdn1_1, Wdn1_2, bdn1_2, Wdn2_1, bdn2_1, Wdn2_2, bdn2_2, Wdn3_1, bdn3_1, Wdn3_2, bdn3_2, Wr_1, br_1, Wr_2, br_2, up2_0, up2_1, up3_0, up3_1, up4_0, up4_1, dn3_0, dn3_1, dn2_0, dn2_1, dn1_0, dn1_1, ring_in, ring_out):
    n1, n2, n3, n4 = h1.shape[0], h2.shape[0], h3.shape[0], h4.shape[0]
    NR = 10000
    h2 = _mlp(jnp.concatenate([h2, _seg(h1, up2_0, n2), _seg(h1, up2_1, n2)], axis=-1), Wup2_1, bup2_1, Wup2_2, bup2_2)
    h3 = _mlp(jnp.concatenate([h3, _seg(h2, up3_0, n3), _seg(h2, up3_1, n3)], axis=-1), Wup3_1, bup3_1, Wup3_2, bup3_2)
    h4 = _mlp(jnp.concatenate([h4, _seg(h3, up4_0, n4), _seg(h3, up4_1, n4)], axis=-1), Wup4_1, bup4_1, Wup4_2, bup4_2)
    h3 = _mlp(jnp.concatenate([h3, _seg(h4, dn3_0, n3) + _seg(h4, dn3_1, n3)], axis=-1), Wdn3_1, bdn3_1, Wdn3_2, bdn3_2)
    h2 = _mlp(jnp.concatenate([h2, _seg(h3, dn2_0, n2) + _seg(h3, dn2_1, n2)], axis=-1), Wdn2_1, bdn2_1, Wdn2_2, bdn2_2)
    h1 = _mlp(jnp.concatenate([h1, _seg(h2, dn1_0, n1) + _seg(h2, dn1_1, n1)], axis=-1), Wdn1_1, bdn1_1, Wdn1_2, bdn1_2)
    hr = _seg(h1, ring_in, NR)
    h_ring_down = _seg(hr, ring_out, n1)
    h1 = _mlp(jnp.concatenate([h1, h_ring_down], axis=-1), Wr_1, br_1, Wr_2, br_2)
    h1 = _identity_pallas(h1)
    return (h1, h2, h3, h4, hr)


# speedup vs baseline: 1.8457x; 1.8457x over previous
import jax
import jax.numpy as jnp
from jax import lax
from jax.experimental import pallas as pl
from jax.experimental.pallas import tpu as pltpu
from jax.experimental.pallas import tpu_sc as plsc

U = 64
NC, NS = 2, 16
TILE = 800
GB = 80
NB = TILE // GB
ZR = 160
BN = 512

_MESH = plsc.VectorSubcoreMesh(
    core_axis_name="c", subcore_axis_name="s", num_cores=NC, num_subcores=NS)
_SC_PARAMS = pltpu.CompilerParams(use_tc_tiling_on_sc=False)

_f32 = jnp.float32


def _acc_geom(n):
    rpw = -(-n // NS)
    rpw = -(-rpw // ZR) * ZR
    return rpw, NS * rpw


def _edge_tiles(e):
    return e[0].reshape(-1, NB, GB), e[1].reshape(-1, NB, GB)


def _fill_zero(zbuf):
    zr, cols = zbuf.shape
    z = jnp.zeros((16,), _f32)

    @pl.loop(0, zr)
    def _(r):
        @pl.loop(0, cols, step=16)
        def _(cc):
            zbuf[r, pl.ds(cc, 16)] = z


def _zero_rows(acc, zbuf, lo, nrows):
    @pl.loop(0, nrows, step=ZR)
    def _(r):
        pltpu.sync_copy(zbuf, acc.at[pl.ds(lo + r, ZR)])


def _stream_tiles(table_ref, src_r, dst_r, acc, sbuf, dbuf, rows, sem, start, step):
    t_tot = src_r.shape[0]

    @pl.loop(start, t_tot, step=step)
    def _(t):
        pltpu.sync_copy(src_r.at[t], sbuf)
        pltpu.sync_copy(dst_r.at[t], dbuf)
        for j in range(NB):
            pltpu.async_copy(table_ref.at[sbuf.at[j]], rows, sem).wait()
            pltpu.sync_copy(rows, acc.at[dbuf.at[j]], add=True)


def _seg_edge_split(table, streams, n, num_accs):
    rpw, npad = _acc_geom(n)
    n_str = len(streams)
    acc_ids = [a for (_, _, a) in streams]

    def body(*refs):
        table_ref = refs[0]
        stream_refs = refs[1:1 + 2 * n_str]
        outs = refs[1 + 2 * n_str:1 + 2 * n_str + num_accs]
        accs = refs[1 + 2 * n_str + num_accs:1 + 2 * n_str + 2 * num_accs]
        sbuf, dbuf, rows, zbuf, sem = refs[1 + 2 * n_str + 2 * num_accs:]
        c = lax.axis_index("c")
        s = lax.axis_index("s")
        gw = c * NS + s
        _fill_zero(zbuf)
        for a in range(num_accs):
            _zero_rows(accs[a], zbuf, s * rpw, rpw)
        plsc.subcore_barrier()
        for k in range(n_str):
            _stream_tiles(table_ref, stream_refs[2 * k], stream_refs[2 * k + 1],
                          accs[acc_ids[k]], sbuf, dbuf, rows, sem,
                          gw, NC * NS)
        plsc.subcore_barrier()
        for a in range(num_accs):
            pltpu.sync_copy(accs[a].at[pl.ds(s * rpw, rpw)],
                            outs[a].at[c, pl.ds(s * rpw, rpw)])

    out_type = [jax.ShapeDtypeStruct((NC, npad, U), _f32) for _ in range(num_accs)]
    scratch = (
        [pltpu.VMEM_SHARED((npad, U), _f32) for _ in range(num_accs)]
        + [pltpu.VMEM((NB, GB), jnp.int32), pltpu.VMEM((NB, GB), jnp.int32),
           pltpu.VMEM((GB, U), _f32), pltpu.VMEM((ZR, U), _f32),
           pltpu.SemaphoreType.DMA])
    fn = pl.kernel(body, out_type=out_type, mesh=_MESH, scratch_types=scratch, compiler_params=_SC_PARAMS)
    args = [table]
    for (sr, dr, _) in streams:
        args += [sr, dr]
    return fn(*args)


def _seg_col_split(tables, passes, n, csz, table_idx, slot_idx, n_slots):
    rpw, npad = _acc_geom(n)
    n_pass = len(passes)
    flat = [st for pstreams in passes for st in pstreams]

    def body(*refs):
        tables_ref = refs[0]
        stream_refs = refs[1:1 + 2 * len(flat)]
        out = refs[1 + 2 * len(flat)]
        acc, sbuf, dbuf, rows, zbuf, sem = refs[2 + 2 * len(flat):]
        c = lax.axis_index("c")
        s = lax.axis_index("s")
        _fill_zero(zbuf)
        off = 0
        for p in range(n_pass):
            _zero_rows(acc, zbuf, s * rpw, rpw)
            plsc.subcore_barrier()
            t_idx = table_idx(c, p)
            slot = slot_idx(c, p)
            for k in range(len(passes[p])):
                _stream_tiles(tables_ref.at[t_idx],
                              stream_refs[2 * (off + k)],
                              stream_refs[2 * (off + k) + 1],
                              acc, sbuf, dbuf, rows, sem, s, NS)
            off += len(passes[p])
            plsc.subcore_barrier()
            pltpu.sync_copy(acc.at[pl.ds(s * rpw, rpw)],
                            out.at[slot, pl.ds(s * rpw, rpw)])

    out_type = jax.ShapeDtypeStruct((n_slots, npad, csz), _f32)
    scratch = [pltpu.VMEM_SHARED((npad, csz), _f32),
               pltpu.VMEM((NB, GB), jnp.int32), pltpu.VMEM((NB, GB), jnp.int32),
               pltpu.VMEM((GB, csz), _f32), pltpu.VMEM((ZR, csz), _f32),
               pltpu.SemaphoreType.DMA]
    fn = pl.kernel(body, out_type=out_type, mesh=_MESH, scratch_types=scratch, compiler_params=_SC_PARAMS)
    args = [tables]
    for (sr, dr) in flat:
        args += [sr, dr]
    return fn(*args)


def _mlp(terms, b1, W2, b2, n, shard_out=None):
    n_t = len(terms)

    def body(*refs):
        t_refs = refs[:n_t]
        w1_refs = refs[n_t:2 * n_t]
        b1_ref, w2_ref, b2_ref = refs[2 * n_t:2 * n_t + 3]
        out_refs = refs[2 * n_t + 3:]
        acc = jnp.broadcast_to(b1_ref[...], (BN, U)).astype(_f32)
        for t, (arr, kind, _) in enumerate(terms):
            if kind == 'full':
                x = t_refs[t][...]
            elif kind == 'pair':
                x = t_refs[t][0] + t_refs[t][1]
            else:
                x = t_refs[t][0]
            acc = acc + jnp.dot(x, w1_refs[t][...], preferred_element_type=_f32)
        h = jnp.tanh(acc)
        y = jnp.dot(h, w2_ref[...], preferred_element_type=_f32) + b2_ref[...]
        out_refs[0][...] = y
        if shard_out is not None:
            qs, csz = shard_out
            for q in range(qs):
                out_refs[1][q] = y[:, q * csz:(q + 1) * csz]

    in_specs = []
    args = []
    for (arr, kind, _) in terms:
        if kind == 'full':
            in_specs.append(pl.BlockSpec((BN, arr.shape[1]), lambda i: (i, 0)))
        elif kind == 'pair':
            in_specs.append(pl.BlockSpec((NC, BN, U), lambda i: (0, i, 0)))
        else:
            _, q, qs = kind
            in_specs.append(pl.BlockSpec((1, BN, arr.shape[2]),
                                         lambda i, q=q: (q, i, 0)))
        args.append(arr)
    for (_, _, w1s) in terms:
        in_specs.append(pl.BlockSpec(w1s.shape, lambda i: (0, 0)))
        args.append(w1s)
    in_specs += [pl.BlockSpec((1, U), lambda i: (0, 0)),
                 pl.BlockSpec((U, U), lambda i: (0, 0)),
                 pl.BlockSpec((1, U), lambda i: (0, 0))]
    args += [b1.reshape(1, U), W2, b2.reshape(1, U)]
    out_shape = [jax.ShapeDtypeStruct((n, U), _f32)]
    out_specs = [pl.BlockSpec((BN, U), lambda i: (i, 0))]
    if shard_out is not None:
        qs, csz = shard_out
        out_shape.append(jax.ShapeDtypeStruct((qs, n, csz), _f32))
        out_specs.append(pl.BlockSpec((qs, BN, csz), lambda i: (0, i, 0)))
    res = pl.pallas_call(
        body, grid=(pl.cdiv(n, BN),),
        in_specs=in_specs, out_specs=out_specs, out_shape=out_shape)(*args)
    return res if shard_out is not None else res[0]


def _assemble_hr(parts, n):
    def body(p_ref, o_ref, q_ref):
        y = p_ref[0] + p_ref[1]
        o_ref[...] = y
        for q in range(4):
            q_ref[q] = y[:, q * 16:(q + 1) * 16]

    return pl.pallas_call(
        body, grid=(pl.cdiv(n, BN),),
        in_specs=[pl.BlockSpec((NC, BN, U), lambda i: (0, i, 0))],
        out_specs=[pl.BlockSpec((BN, U), lambda i: (i, 0)),
                   pl.BlockSpec((4, BN, 16), lambda i: (0, i, 0))],
        out_shape=[jax.ShapeDtypeStruct((n, U), _f32),
                   jax.ShapeDtypeStruct((4, n, 16), _f32)])(parts)


def kernel(h1, h2, h3, h4, Wup2_1, bup2_1, Wup2_2, bup2_2, Wup3_1, bup3_1, Wup3_2, bup3_2, Wup4_1, bup4_1, Wup4_2, bup4_2, Wdn1_1, bdn1_1, Wdn1_2, bdn1_2, Wdn2_1, bdn2_1, Wdn2_2, bdn2_2, Wdn3_1, bdn3_1, Wdn3_2, bdn3_2, Wr_1, br_1, Wr_2, br_2, up2_0, up2_1, up3_0, up3_1, up4_0, up4_1, dn3_0, dn3_1, dn2_0, dn2_1, dn1_0, dn1_1, ring_in, ring_out):
    n1, n2, n3, n4 = h1.shape[0], h2.shape[0], h3.shape[0], h4.shape[0]
    NR = 10000

    u20, u21 = _edge_tiles(up2_0), _edge_tiles(up2_1)
    u30, u31 = _edge_tiles(up3_0), _edge_tiles(up3_1)
    u40, u41 = _edge_tiles(up4_0), _edge_tiles(up4_1)
    d30, d31 = _edge_tiles(dn3_0), _edge_tiles(dn3_1)
    d20, d21 = _edge_tiles(dn2_0), _edge_tiles(dn2_1)
    d10, d11 = _edge_tiles(dn1_0), _edge_tiles(dn1_1)
    rin, rout = _edge_tiles(ring_in), _edge_tiles(ring_out)

    h1c = jnp.stack([h1[:, :32], h1[:, 32:]])
    s2 = _seg_col_split(
        h1c, [[u20], [u21]], n2, 32,
        table_idx=lambda c, p: c, slot_idx=lambda c, p: 2 * p + c, n_slots=4)
    h2u = _mlp([(h2, 'full', Wup2_1[0:64]),
                (s2, ('slot', 0, 4), Wup2_1[64:96]),
                (s2, ('slot', 1, 4), Wup2_1[96:128]),
                (s2, ('slot', 2, 4), Wup2_1[128:160]),
                (s2, ('slot', 3, 4), Wup2_1[160:192])],
               bup2_1, Wup2_2, bup2_2, n2)

    s30, s31 = _seg_edge_split(h2u, [(u30[0], u30[1], 0), (u31[0], u31[1], 1)],
                               n3, num_accs=2)
    h3u = _mlp([(h3, 'full', Wup3_1[0:64]),
                (s30, 'pair', Wup3_1[64:128]),
                (s31, 'pair', Wup3_1[128:192])],
               bup3_1, Wup3_2, bup3_2, n3)

    s40, s41 = _seg_edge_split(h3u, [(u40[0], u40[1], 0), (u41[0], u41[1], 1)],
                               n4, num_accs=2)
    h4f = _mlp([(h4, 'full', Wup4_1[0:64]),
                (s40, 'pair', Wup4_1[64:128]),
                (s41, 'pair', Wup4_1[128:192])],
               bup4_1, Wup4_2, bup4_2, n4)

    (d3,) = _seg_edge_split(h4f, [(d30[0], d30[1], 0), (d31[0], d31[1], 0)],
                            n3, num_accs=1)
    h3f, h3f_c = _mlp([(h3u, 'full', Wdn3_1[0:64]),
                       (d3, 'pair', Wdn3_1[64:128])],
                      bdn3_1, Wdn3_2, bdn3_2, n3, shard_out=(2, 32))

    d2 = _seg_col_split(
        h3f_c, [[d20, d21]], n2, 32,
        table_idx=lambda c, p: c, slot_idx=lambda c, p: c, n_slots=2)
    h2f, h2f_q = _mlp([(h2u, 'full', Wdn2_1[0:64]),
                       (d2, ('slot', 0, 2), Wdn2_1[64:96]),
                       (d2, ('slot', 1, 2), Wdn2_1[96:128])],
                      bdn2_1, Wdn2_2, bdn2_2, n2, shard_out=(4, 16))

    d1 = _seg_col_split(
        h2f_q, [[d10, d11], [d10, d11]], n1, 16,
        table_idx=lambda c, p: 2 * c + p, slot_idx=lambda c, p: 2 * c + p,
        n_slots=4)
    h1r = _mlp([(h1, 'full', Wdn1_1[0:64]),
                (d1, ('slot', 0, 4), Wdn1_1[64:80]),
                (d1, ('slot', 1, 4), Wdn1_1[80:96]),
                (d1, ('slot', 2, 4), Wdn1_1[96:112]),
                (d1, ('slot', 3, 4), Wdn1_1[112:128])],
               bdn1_1, Wdn1_2, bdn1_2, n1)

    (hrp,) = _seg_edge_split(h1r, [(rin[0], rin[1], 0)], NR, num_accs=1)
    hr, hr_q = _assemble_hr(hrp, NR)
    r1 = _seg_col_split(
        hr_q, [[rout], [rout]], n1, 16,
        table_idx=lambda c, p: 2 * c + p, slot_idx=lambda c, p: 2 * c + p,
        n_slots=4)
    h1f = _mlp([(h1r, 'full', Wr_1[0:64]),
                (r1, ('slot', 0, 4), Wr_1[64:80]),
                (r1, ('slot', 1, 4), Wr_1[80:96]),
                (r1, ('slot', 2, 4), Wr_1[96:112]),
                (r1, ('slot', 3, 4), Wr_1[112:128])],
               br_1, Wr_2, br_2, n1)

    return (h1f, h2f, h3f, h4f, hr)


# speedup vs baseline: 2.6101x; 1.4142x over previous
import jax
import jax.numpy as jnp
from jax import lax
from jax.experimental import pallas as pl
from jax.experimental.pallas import tpu as pltpu
from jax.experimental.pallas import tpu_sc as plsc

U = 64
NC, NS = 2, 16
GB = 128
KPIPE = 4
BN = 512

_MESH = plsc.VectorSubcoreMesh(
    core_axis_name="c", subcore_axis_name="s", num_cores=NC, num_subcores=NS)
_SC_PARAMS = pltpu.CompilerParams(use_tc_tiling_on_sc=False)

_f32 = jnp.float32


def _zr(csz):
    return (8 * 1024) // (4 * csz)


def _acc_geom(n, csz):
    zr = _zr(csz)
    rpw = -(-n // NS)
    rpw = -(-rpw // zr) * zr
    return rpw, NS * rpw


def _edge_tiles(e, nb, n_dst):
    E = e.shape[1]
    tile = nb * GB
    Ep = -(-E // tile) * tile
    src, dst = e[0], e[1]
    if Ep != E:
        src = jnp.concatenate([src, jnp.zeros((Ep - E,), e.dtype)])
        dst = jnp.concatenate([dst, jnp.full((Ep - E,), n_dst, e.dtype)])
    return src.reshape(-1, nb, GB), dst.reshape(-1, nb, GB)


def _fill_zero(zbuf):
    zr, cols = zbuf.shape
    z = jnp.zeros((16,), _f32)

    @pl.loop(0, zr)
    def _(r):
        @pl.loop(0, cols, step=16)
        def _(cc):
            zbuf[r, pl.ds(cc, 16)] = z


def _zero_rows(acc, zbuf, lo, nrows, sem):
    zr = zbuf.shape[0]
    nz = nrows // zr

    @pl.loop(0, nz)
    def _(i):
        pltpu.async_copy(zbuf, acc.at[pl.ds(lo + i * zr, zr)], sem)

    @pl.loop(0, nz)
    def _(i):
        pltpu.make_async_copy(zbuf, acc.at[pl.ds(lo, zr)], sem).wait()


def _stream_tiles(table_ref, src_r, dst_r, acc, sbuf, dbuf, rowbufs, gsems,
                  ssems, start, step):
    t_tot, nb = src_r.shape[0], src_r.shape[1]

    D = KPIPE // 2

    @pl.loop(start, t_tot, step=step)
    def _(t):
        pltpu.sync_copy(src_r.at[t], sbuf)
        pltpu.sync_copy(dst_r.at[t], dbuf)
        for j in range(min(D, nb)):
            pltpu.async_copy(table_ref.at[sbuf.at[j]], rowbufs[j % KPIPE],
                             gsems[j % KPIPE])
        for j in range(nb):
            k = j % KPIPE
            if j + D < nb:
                kd = (j + D) % KPIPE
                jprev = j + D - KPIPE
                if jprev >= 0:
                    pltpu.make_async_copy(rowbufs[kd], acc.at[dbuf.at[jprev]],
                                          ssems[kd]).wait()
                pltpu.async_copy(table_ref.at[sbuf.at[j + D]], rowbufs[kd],
                                 gsems[kd])
            pltpu.make_async_copy(table_ref.at[sbuf.at[j]], rowbufs[k],
                                  gsems[k]).wait()
            pltpu.async_copy(rowbufs[k], acc.at[dbuf.at[j]], ssems[k],
                             add=True)
        for j in range(max(0, nb - KPIPE), nb):
            k = j % KPIPE
            pltpu.make_async_copy(rowbufs[k], acc.at[dbuf.at[j]],
                                  ssems[k]).wait()


def _sc_scratch(npad, csz, nb, num_accs):
    return ([pltpu.VMEM_SHARED((npad, csz), _f32) for _ in range(num_accs)]
            + [pltpu.VMEM((nb, GB), jnp.int32),
               pltpu.VMEM((nb, GB), jnp.int32)]
            + [pltpu.VMEM((GB, csz), _f32) for _ in range(KPIPE)]
            + [pltpu.VMEM((_zr(csz), csz), _f32)]
            + [pltpu.SemaphoreType.DMA] * (2 * KPIPE + 1))


def _seg_edge_split(table, streams, n, num_accs):
    rpw, npad = _acc_geom(n, U)
    n_str = len(streams)
    acc_ids = [a for (_, _, a) in streams]
    nb = streams[0][0].shape[1]

    def body(*refs):
        table_ref = refs[0]
        stream_refs = refs[1:1 + 2 * n_str]
        outs = refs[1 + 2 * n_str:1 + 2 * n_str + num_accs]
        rest = refs[1 + 2 * n_str + num_accs:]
        accs = rest[:num_accs]
        sbuf, dbuf = rest[num_accs:num_accs + 2]
        rowbufs = rest[num_accs + 2:num_accs + 2 + KPIPE]
        zbuf = rest[num_accs + 2 + KPIPE]
        gsems = rest[num_accs + 3 + KPIPE:num_accs + 3 + 2 * KPIPE]
        ssems = rest[num_accs + 3 + 2 * KPIPE:num_accs + 3 + 3 * KPIPE]
        zsem = rest[num_accs + 3 + 3 * KPIPE]
        c = lax.axis_index("c")
        s = lax.axis_index("s")
        gw = c * NS + s
        _fill_zero(zbuf)
        for a in range(num_accs):
            _zero_rows(accs[a], zbuf, s * rpw, rpw, zsem)
        plsc.subcore_barrier()
        for k in range(n_str):
            _stream_tiles(table_ref, stream_refs[2 * k], stream_refs[2 * k + 1],
                          accs[acc_ids[k]], sbuf, dbuf, rowbufs, gsems, ssems,
                          gw, NC * NS)
        plsc.subcore_barrier()
        for a in range(num_accs):
            pltpu.sync_copy(accs[a].at[pl.ds(s * rpw, rpw)],
                            outs[a].at[c, pl.ds(s * rpw, rpw)])

    out_type = [jax.ShapeDtypeStruct((NC, npad, U), _f32) for _ in range(num_accs)]
    fn = pl.kernel(body, out_type=out_type, mesh=_MESH,
                   scratch_types=_sc_scratch(npad, U, nb, num_accs),
                   compiler_params=_SC_PARAMS)
    args = [table]
    for (sr, dr, _) in streams:
        args += [sr, dr]
    return fn(*args)


def _seg_col_split(tables, passes, n, csz, table_idx, slot_idx, n_slots,
                   stream_by_core=False):
    rpw, npad = _acc_geom(n, csz)
    n_pass = len(passes)
    flat = [st for pstreams in passes for st in pstreams]
    nb = flat[0][0].shape[1]

    def body(*refs):
        tables_ref = refs[0]
        stream_refs = refs[1:1 + 2 * len(flat)]
        out = refs[1 + 2 * len(flat)]
        rest = refs[2 + 2 * len(flat):]
        acc, sbuf, dbuf = rest[:3]
        rowbufs = rest[3:3 + KPIPE]
        zbuf = rest[3 + KPIPE]
        gsems = rest[4 + KPIPE:4 + 2 * KPIPE]
        ssems = rest[4 + 2 * KPIPE:4 + 3 * KPIPE]
        zsem = rest[4 + 3 * KPIPE]
        c = lax.axis_index("c")
        s = lax.axis_index("s")
        _fill_zero(zbuf)
        off = 0
        for p in range(n_pass):
            _zero_rows(acc, zbuf, s * rpw, rpw, zsem)
            plsc.subcore_barrier()
            t_idx = table_idx(c, p)
            slot = slot_idx(c, p)
            for k in range(len(passes[p])):
                def _run(k=k):
                    _stream_tiles(tables_ref.at[t_idx],
                                  stream_refs[2 * (off + k)],
                                  stream_refs[2 * (off + k) + 1],
                                  acc, sbuf, dbuf, rowbufs, gsems, ssems,
                                  s, NS)
                if stream_by_core:
                    pl.when(c == k)(_run)
                else:
                    _run()
            off += len(passes[p])
            plsc.subcore_barrier()
            pltpu.sync_copy(acc.at[pl.ds(s * rpw, rpw)],
                            out.at[slot, pl.ds(s * rpw, rpw)])

    out_type = jax.ShapeDtypeStruct((n_slots, npad, csz), _f32)
    fn = pl.kernel(body, out_type=out_type, mesh=_MESH,
                   scratch_types=_sc_scratch(npad, csz, nb, 1),
                   compiler_params=_SC_PARAMS)
    args = [tables]
    for (sr, dr) in flat:
        args += [sr, dr]
    return fn(*args)


def _mlp(terms, b1, W2, b2, n, shard_out=None):
    n_t = len(terms)

    def body(*refs):
        t_refs = refs[:n_t]
        w1_refs = refs[n_t:2 * n_t]
        b1_ref, w2_ref, b2_ref = refs[2 * n_t:2 * n_t + 3]
        out_refs = refs[2 * n_t + 3:]
        acc = jnp.broadcast_to(b1_ref[...], (BN, U)).astype(_f32)
        for t, (arr, kind, _) in enumerate(terms):
            if kind == 'full':
                x = t_refs[t][...]
            elif kind == 'pair':
                x = t_refs[t][0] + t_refs[t][1]
            else:
                x = t_refs[t][0]
            acc = acc + jnp.dot(x, w1_refs[t][...], preferred_element_type=_f32)
        h = jnp.tanh(acc)
        y = jnp.dot(h, w2_ref[...], preferred_element_type=_f32) + b2_ref[...]
        out_refs[0][...] = y
        if shard_out is not None:
            qs, csz = shard_out
            for q in range(qs):
                out_refs[1][q] = y[:, q * csz:(q + 1) * csz]

    in_specs = []
    args = []
    for (arr, kind, _) in terms:
        if kind == 'full':
            in_specs.append(pl.BlockSpec((BN, arr.shape[1]), lambda i: (i, 0)))
        elif kind == 'pair':
            in_specs.append(pl.BlockSpec((NC, BN, U), lambda i: (0, i, 0)))
        else:
            _, q, qs = kind
            in_specs.append(pl.BlockSpec((1, BN, arr.shape[2]),
                                         lambda i, q=q: (q, i, 0)))
        args.append(arr)
    for (_, _, w1s) in terms:
        in_specs.append(pl.BlockSpec(w1s.shape, lambda i: (0, 0)))
        args.append(w1s)
    in_specs += [pl.BlockSpec((1, U), lambda i: (0, 0)),
                 pl.BlockSpec((U, U), lambda i: (0, 0)),
                 pl.BlockSpec((1, U), lambda i: (0, 0))]
    args += [b1.reshape(1, U), W2, b2.reshape(1, U)]
    out_shape = [jax.ShapeDtypeStruct((n, U), _f32)]
    out_specs = [pl.BlockSpec((BN, U), lambda i: (i, 0))]
    if shard_out is not None:
        qs, csz = shard_out
        out_shape.append(jax.ShapeDtypeStruct((qs, n, csz), _f32))
        out_specs.append(pl.BlockSpec((qs, BN, csz), lambda i: (0, i, 0)))
    res = pl.pallas_call(
        body, grid=(pl.cdiv(n, BN),),
        in_specs=in_specs, out_specs=out_specs, out_shape=out_shape)(*args)
    return res if shard_out is not None else res[0]


def _assemble_hr(parts, n):
    def body(p_ref, o_ref, q_ref):
        y = p_ref[0] + p_ref[1]
        o_ref[...] = y
        for q in range(4):
            q_ref[q] = y[:, q * 16:(q + 1) * 16]

    return pl.pallas_call(
        body, grid=(pl.cdiv(n, BN),),
        in_specs=[pl.BlockSpec((NC, BN, U), lambda i: (0, i, 0))],
        out_specs=[pl.BlockSpec((BN, U), lambda i: (i, 0)),
                   pl.BlockSpec((4, BN, 16), lambda i: (0, i, 0))],
        out_shape=[jax.ShapeDtypeStruct((n, U), _f32),
                   jax.ShapeDtypeStruct((4, n, 16), _f32)])(parts)


def kernel(h1, h2, h3, h4, Wup2_1, bup2_1, Wup2_2, bup2_2, Wup3_1, bup3_1, Wup3_2, bup3_2, Wup4_1, bup4_1, Wup4_2, bup4_2, Wdn1_1, bdn1_1, Wdn1_2, bdn1_2, Wdn2_1, bdn2_1, Wdn2_2, bdn2_2, Wdn3_1, bdn3_1, Wdn3_2, bdn3_2, Wr_1, br_1, Wr_2, br_2, up2_0, up2_1, up3_0, up3_1, up4_0, up4_1, dn3_0, dn3_1, dn2_0, dn2_1, dn1_0, dn1_1, ring_in, ring_out):
    n1, n2, n3, n4 = h1.shape[0], h2.shape[0], h3.shape[0], h4.shape[0]
    NR = 10000

    u20, u21 = _edge_tiles(up2_0, 16, n2), _edge_tiles(up2_1, 16, n2)
    u30, u31 = _edge_tiles(up3_0, 8, n3), _edge_tiles(up3_1, 8, n3)
    u40, u41 = _edge_tiles(up4_0, 4, n4), _edge_tiles(up4_1, 4, n4)
    d30, d31 = _edge_tiles(dn3_0, 4, n3), _edge_tiles(dn3_1, 4, n3)
    d20, d21 = _edge_tiles(dn2_0, 8, n2), _edge_tiles(dn2_1, 8, n2)
    d10, d11 = _edge_tiles(dn1_0, 16, n1), _edge_tiles(dn1_1, 16, n1)
    rin, rout = _edge_tiles(ring_in, 8, NR), _edge_tiles(ring_out, 8, n1)

    h1c = jnp.stack([h1[:, :32], h1[:, 32:]])
    s2 = _seg_col_split(
        h1c, [[u20], [u21]], n2, 32,
        table_idx=lambda c, p: c, slot_idx=lambda c, p: 2 * p + c, n_slots=4)
    h2u = _mlp([(h2, 'full', Wup2_1[0:64]),
                (s2, ('slot', 0, 4), Wup2_1[64:96]),
                (s2, ('slot', 1, 4), Wup2_1[96:128]),
                (s2, ('slot', 2, 4), Wup2_1[128:160]),
                (s2, ('slot', 3, 4), Wup2_1[160:192])],
               bup2_1, Wup2_2, bup2_2, n2)

    s3 = _seg_col_split(
        h2u.reshape(1, n2, U), [[u30, u31]], n3, U,
        table_idx=lambda c, p: 0, slot_idx=lambda c, p: c, n_slots=2,
        stream_by_core=True)
    h3u = _mlp([(h3, 'full', Wup3_1[0:64]),
                (s3, ('slot', 0, 2), Wup3_1[64:128]),
                (s3, ('slot', 1, 2), Wup3_1[128:192])],
               bup3_1, Wup3_2, bup3_2, n3)

    s4 = _seg_col_split(
        h3u.reshape(1, n3, U), [[u40, u41]], n4, U,
        table_idx=lambda c, p: 0, slot_idx=lambda c, p: c, n_slots=2,
        stream_by_core=True)
    h4f = _mlp([(h4, 'full', Wup4_1[0:64]),
                (s4, ('slot', 0, 2), Wup4_1[64:128]),
                (s4, ('slot', 1, 2), Wup4_1[128:192])],
               bup4_1, Wup4_2, bup4_2, n4)

    (d3,) = _seg_edge_split(h4f, [(d30[0], d30[1], 0), (d31[0], d31[1], 0)],
                            n3, num_accs=1)
    h3f, h3f_c = _mlp([(h3u, 'full', Wdn3_1[0:64]),
                       (d3, 'pair', Wdn3_1[64:128])],
                      bdn3_1, Wdn3_2, bdn3_2, n3, shard_out=(2, 32))

    d2 = _seg_col_split(
        h3f_c, [[d20, d21]], n2, 32,
        table_idx=lambda c, p: c, slot_idx=lambda c, p: c, n_slots=2)
    h2f, h2f_q = _mlp([(h2u, 'full', Wdn2_1[0:64]),
                       (d2, ('slot', 0, 2), Wdn2_1[64:96]),
                       (d2, ('slot', 1, 2), Wdn2_1[96:128])],
                      bdn2_1, Wdn2_2, bdn2_2, n2, shard_out=(4, 16))

    d1 = _seg_col_split(
        h2f_q, [[d10, d11], [d10, d11]], n1, 16,
        table_idx=lambda c, p: 2 * c + p, slot_idx=lambda c, p: 2 * c + p,
        n_slots=4)
    h1r = _mlp([(h1, 'full', Wdn1_1[0:64]),
                (d1, ('slot', 0, 4), Wdn1_1[64:80]),
                (d1, ('slot', 1, 4), Wdn1_1[80:96]),
                (d1, ('slot', 2, 4), Wdn1_1[96:112]),
                (d1, ('slot', 3, 4), Wdn1_1[112:128])],
               bdn1_1, Wdn1_2, bdn1_2, n1)

    (hrp,) = _seg_edge_split(h1r, [(rin[0], rin[1], 0)], NR, num_accs=1)
    hr, hr_q = _assemble_hr(hrp, NR)
    r1 = _seg_col_split(
        hr_q, [[rout], [rout]], n1, 16,
        table_idx=lambda c, p: 2 * c + p, slot_idx=lambda c, p: 2 * c + p,
        n_slots=4)
    h1f = _mlp([(h1r, 'full', Wr_1[0:64]),
                (r1, ('slot', 0, 4), Wr_1[64:80]),
                (r1, ('slot', 1, 4), Wr_1[80:96]),
                (r1, ('slot', 2, 4), Wr_1[96:112]),
                (r1, ('slot', 3, 4), Wr_1[112:128])],
               br_1, Wr_2, br_2, n1)

    return (h1f, h2f, h3f, h4f, hr)


# speedup vs baseline: 3.3312x; 1.2762x over previous
import jax
import jax.numpy as jnp
from jax import lax
from jax.experimental import pallas as pl
from jax.experimental.pallas import tpu as pltpu
from jax.experimental.pallas import tpu_sc as plsc

U = 64
NC, NS = 2, 16
GB = 128
KPIPE = 4
BN = 512
ZR = 32
NDUMP = 64

_MESH = plsc.VectorSubcoreMesh(
    core_axis_name="c", subcore_axis_name="s", num_cores=NC, num_subcores=NS)
_SC_PARAMS = pltpu.CompilerParams(use_tc_tiling_on_sc=False)

_f32 = jnp.float32


def _acc_geom(n):
    rpw = -(-n // NS)
    rpw = -(-rpw // ZR) * ZR
    return rpw, NS * rpw


def _chunk_size(n, n_chunks):
    return -(-(-(-n // n_chunks)) // (NS * ZR)) * (NS * ZR)


def _edge_tiles(e, nb, n_dst, gb=GB):
    E = e.shape[1]
    tile = nb * gb
    Ep = -(-E // tile) * tile
    src, dst = e[0], e[1]
    if Ep != E:
        src = jnp.concatenate([src, jnp.zeros((Ep - E,), e.dtype)])
        dst = jnp.concatenate([dst, jnp.full((Ep - E,), n_dst, e.dtype)])
    return src.reshape(-1, nb, gb), dst.reshape(-1, nb, gb)


def _fill_zero(zbuf):
    zr, cols = zbuf.shape
    z = jnp.zeros((16,), _f32)

    @pl.loop(0, zr)
    def _(r):
        @pl.loop(0, cols, step=16)
        def _(cc):
            zbuf[r, pl.ds(cc, 16)] = z


def _zero_rows(acc, zbuf, lo, nrows, sem):
    nz = nrows // ZR

    @pl.loop(0, nz)
    def _(i):
        pltpu.async_copy(zbuf, acc.at[pl.ds(lo + i * ZR, ZR)], sem)

    @pl.loop(0, nz)
    def _(i):
        pltpu.make_async_copy(zbuf, acc.at[pl.ds(lo, ZR)], sem).wait()


def _stream_tiles(table_ref, src_r, dst_r, acc, sbuf, dbuf, rowbufs, gsems,
                  ssems, start, step, clamp=None):
    t_tot, nb, gb = src_r.shape

    D = KPIPE // 2

    @pl.loop(start, t_tot, step=step)
    def _(t):
        pltpu.sync_copy(src_r.at[t], sbuf)
        pltpu.sync_copy(dst_r.at[t], dbuf)
        if clamp is not None:
            lo, cs = clamp

            @pl.loop(0, nb)
            def _(jv):
                for m in range(gb // 16):
                    v = dbuf[jv, pl.ds(16 * m, 16)]
                    inr = (v >= lo) & (v < lo + cs)
                    dump = cs + (v & (NDUMP - 1))
                    dbuf[jv, pl.ds(16 * m, 16)] = jnp.where(inr, v - lo, dump)
        for j in range(min(D, nb)):
            pltpu.async_copy(table_ref.at[sbuf.at[j]], rowbufs[j % KPIPE],
                             gsems[j % KPIPE])
        for j in range(nb):
            k = j % KPIPE
            if j + D < nb:
                kd = (j + D) % KPIPE
                jprev = j + D - KPIPE
                if jprev >= 0:
                    pltpu.make_async_copy(rowbufs[kd], acc.at[dbuf.at[jprev]],
                                          ssems[kd]).wait()
                pltpu.async_copy(table_ref.at[sbuf.at[j + D]], rowbufs[kd],
                                 gsems[kd])
            pltpu.make_async_copy(table_ref.at[sbuf.at[j]], rowbufs[k],
                                  gsems[k]).wait()
            pltpu.async_copy(rowbufs[k], acc.at[dbuf.at[j]], ssems[k],
                             add=True)
        for j in range(max(0, nb - KPIPE), nb):
            k = j % KPIPE
            pltpu.make_async_copy(rowbufs[k], acc.at[dbuf.at[j]],
                                  ssems[k]).wait()


def _sc_scratch(acc_rows, nb, gb):
    return [pltpu.VMEM_SHARED((acc_rows, U), _f32),
            pltpu.VMEM((nb, gb), jnp.int32),
            pltpu.VMEM((nb, gb), jnp.int32)] \
        + [pltpu.VMEM((gb, U), _f32) for _ in range(KPIPE)] \
        + [pltpu.VMEM((ZR, U), _f32)] \
        + [pltpu.SemaphoreType.DMA] * (2 * KPIPE + 1)


def _unpack_refs(refs, n_stream_args):
    stream_refs = refs[:n_stream_args]
    rest = refs[n_stream_args:]
    acc, sbuf, dbuf = rest[:3]
    rowbufs = rest[3:3 + KPIPE]
    zbuf = rest[3 + KPIPE]
    gsems = rest[4 + KPIPE:4 + 2 * KPIPE]
    ssems = rest[4 + 2 * KPIPE:4 + 3 * KPIPE]
    zsem = rest[4 + 3 * KPIPE]
    return stream_refs, acc, sbuf, dbuf, rowbufs, zbuf, gsems, ssems, zsem


def _seg_edge_split(table, streams, n):
    rpw, npad = _acc_geom(n)
    n_str = len(streams)
    nb = streams[0][0].shape[1]

    def body(*refs):
        (srefs, acc, sbuf, dbuf, rowbufs, zbuf, gsems, ssems,
         zsem) = _unpack_refs(refs[1:1 + 2 * n_str] + refs[2 + 2 * n_str:],
                              2 * n_str)
        table_ref = refs[0]
        out = refs[1 + 2 * n_str]
        c = lax.axis_index("c")
        s = lax.axis_index("s")
        gw = c * NS + s
        _fill_zero(zbuf)
        _zero_rows(acc, zbuf, s * rpw, rpw, zsem)
        plsc.subcore_barrier()
        for k in range(n_str):
            _stream_tiles(table_ref, srefs[2 * k], srefs[2 * k + 1],
                          acc, sbuf, dbuf, rowbufs, gsems, ssems, gw, NC * NS)
        plsc.subcore_barrier()
        pltpu.sync_copy(acc.at[pl.ds(s * rpw, rpw)],
                        out.at[c, pl.ds(s * rpw, rpw)])

    out_type = jax.ShapeDtypeStruct((NC, npad, U), _f32)
    fn = pl.kernel(body, out_type=out_type, mesh=_MESH,
                   scratch_types=_sc_scratch(npad + ZR, nb,
                                             streams[0][0].shape[2]),
                   compiler_params=_SC_PARAMS)
    args = [table]
    for (sr, dr) in streams:
        args += [sr, dr]
    return fn(*args)


def _seg_etype_split(table, streams, n):
    rpw, npad = _acc_geom(n)
    nb = streams[0][0].shape[1]

    def body(*refs):
        (srefs, acc, sbuf, dbuf, rowbufs, zbuf, gsems, ssems,
         zsem) = _unpack_refs(refs[1:5] + refs[6:], 4)
        table_ref = refs[0]
        out = refs[5]
        c = lax.axis_index("c")
        s = lax.axis_index("s")
        _fill_zero(zbuf)
        _zero_rows(acc, zbuf, s * rpw, rpw, zsem)
        plsc.subcore_barrier()
        for k in range(2):
            @pl.when(c == k)
            def _(k=k):
                _stream_tiles(table_ref, srefs[2 * k], srefs[2 * k + 1],
                              acc, sbuf, dbuf, rowbufs, gsems, ssems, s, NS)
        plsc.subcore_barrier()
        pltpu.sync_copy(acc.at[pl.ds(s * rpw, rpw)],
                        out.at[c, pl.ds(s * rpw, rpw)])

    out_type = jax.ShapeDtypeStruct((2, npad, U), _f32)
    fn = pl.kernel(body, out_type=out_type, mesh=_MESH,
                   scratch_types=_sc_scratch(npad + ZR, nb,
                                             streams[0][0].shape[2]),
                   compiler_params=_SC_PARAMS)
    return fn(table, streams[0][0], streams[0][1], streams[1][0], streams[1][1])


def _seg_row_split(table, passes, n, n_chunks, chunk_of, out_of, n_out):
    CS = _chunk_size(n, n_chunks)
    rpw_c = CS // NS
    n_pass = len(passes)
    flat = [st for ps in passes for st in ps]
    nb = flat[0][0].shape[1]

    def body(*refs):
        (srefs, acc, sbuf, dbuf, rowbufs, zbuf, gsems, ssems,
         zsem) = _unpack_refs(refs[1:1 + 2 * len(flat)]
                              + refs[2 + 2 * len(flat):], 2 * len(flat))
        table_ref = refs[0]
        out = refs[1 + 2 * len(flat)]
        c = lax.axis_index("c")
        s = lax.axis_index("s")
        _fill_zero(zbuf)
        off = 0
        for p in range(n_pass):
            _zero_rows(acc, zbuf, s * rpw_c, rpw_c, zsem)
            plsc.subcore_barrier()
            q = chunk_of(c, p)
            oidx = out_of(c, p)
            lo = q * CS
            for k in range(len(passes[p])):
                _stream_tiles(table_ref, srefs[2 * (off + k)],
                              srefs[2 * (off + k) + 1],
                              acc, sbuf, dbuf, rowbufs, gsems, ssems,
                              s, NS, clamp=(lo, CS))
            off += len(passes[p])
            plsc.subcore_barrier()
            pltpu.sync_copy(acc.at[pl.ds(s * rpw_c, rpw_c)],
                            out.at[oidx, pl.ds(lo + s * rpw_c, rpw_c)])

    out_type = jax.ShapeDtypeStruct((n_out, n_chunks * CS, U), _f32)
    fn = pl.kernel(body, out_type=out_type, mesh=_MESH,
                   scratch_types=_sc_scratch(CS + NDUMP, nb,
                                             flat[0][0].shape[2]),
                   compiler_params=_SC_PARAMS)
    args = [table]
    for (sr, dr) in flat:
        args += [sr, dr]
    return fn(*args)


def _bd(W):
    k = W.shape[0]
    z = jnp.zeros((k, U), _f32)
    return jnp.concatenate(
        [jnp.concatenate([W, z], axis=1), jnp.concatenate([z, W], axis=1)],
        axis=0)


def _b2x(b):
    return jnp.concatenate([b, b]).reshape(1, 2 * U)


def _mlp(terms, b1, W2, b2, n):
    n_t = len(terms)

    def body(*refs):
        t_refs = refs[:n_t]
        w1_refs = refs[n_t:2 * n_t]
        b1_ref, w2_ref, b2_ref = refs[2 * n_t:2 * n_t + 3]
        o_ref = refs[2 * n_t + 3]
        acc = jnp.broadcast_to(b1_ref[...], (BN // 2, 2 * U)).astype(_f32)
        for t, (arr, kind, _) in enumerate(terms):
            if kind == 'packed':
                x = t_refs[t][...]
            elif kind[0] == 'pairp':
                x = t_refs[t][0] + t_refs[t][1]
            else:
                x = t_refs[t][0]
            acc = acc + jnp.dot(x, w1_refs[t][...], preferred_element_type=_f32)
        h = jnp.tanh(acc)
        o_ref[...] = jnp.dot(h, w2_ref[...],
                             preferred_element_type=_f32) + b2_ref[...]

    in_specs = []
    args = []
    for (arr, kind, _) in terms:
        if kind == 'packed':
            in_specs.append(pl.BlockSpec((BN // 2, 128), lambda i: (i, 0)))
        elif kind[0] == 'pairp':
            in_specs.append(pl.BlockSpec((NC, BN // 2, 128),
                                         lambda i: (0, i, 0)))
        else:
            _, q, qs = kind
            in_specs.append(pl.BlockSpec((1, BN // 2, 128),
                                         lambda i, q=q: (q, i, 0)))
        args.append(arr)
    for (_, _, w1s) in terms:
        in_specs.append(pl.BlockSpec((128, 128), lambda i: (0, 0)))
        args.append(_bd(w1s))
    in_specs += [pl.BlockSpec((1, 128), lambda i: (0, 0)),
                 pl.BlockSpec((128, 128), lambda i: (0, 0)),
                 pl.BlockSpec((1, 128), lambda i: (0, 0))]
    args += [_b2x(b1), _bd(W2), _b2x(b2)]
    return pl.pallas_call(
        body, grid=(pl.cdiv(n, BN),),
        in_specs=in_specs,
        out_specs=pl.BlockSpec((BN // 2, 128), lambda i: (i, 0)),
        out_shape=jax.ShapeDtypeStruct((n * U // 128, 128), _f32))(*args)


def _pair_sum(parts, n):
    def body(p_ref, o_ref):
        o_ref[...] = p_ref[0] + p_ref[1]

    return pl.pallas_call(
        body, grid=(pl.cdiv(n, BN),),
        in_specs=[pl.BlockSpec((NC, BN // 2, 128), lambda i: (0, i, 0))],
        out_specs=pl.BlockSpec((BN // 2, 128), lambda i: (i, 0)),
        out_shape=jax.ShapeDtypeStruct((n * U // 128, 128), _f32))(parts)


def _pack2(x):
    n = x.shape[0]
    return lax.optimization_barrier(x.reshape(n * U // 128, 128))


def _pku(x, n):
    return x.reshape(-1, U)[:n]


def kernel(h1, h2, h3, h4, Wup2_1, bup2_1, Wup2_2, bup2_2, Wup3_1, bup3_1, Wup3_2, bup3_2, Wup4_1, bup4_1, Wup4_2, bup4_2, Wdn1_1, bdn1_1, Wdn1_2, bdn1_2, Wdn2_1, bdn2_1, Wdn2_2, bdn2_2, Wdn3_1, bdn3_1, Wdn3_2, bdn3_2, Wr_1, br_1, Wr_2, br_2, up2_0, up2_1, up3_0, up3_1, up4_0, up4_1, dn3_0, dn3_1, dn2_0, dn2_1, dn1_0, dn1_1, ring_in, ring_out):
    n1, n2, n3, n4 = h1.shape[0], h2.shape[0], h3.shape[0], h4.shape[0]
    NR = 10000

    u20, u21 = _edge_tiles(up2_0, 32, n2, 64), _edge_tiles(up2_1, 32, n2, 64)
    u30, u31 = _edge_tiles(up3_0, 8, n3), _edge_tiles(up3_1, 8, n3)
    u40, u41 = _edge_tiles(up4_0, 4, n4), _edge_tiles(up4_1, 4, n4)
    d30, d31 = _edge_tiles(dn3_0, 4, n3), _edge_tiles(dn3_1, 4, n3)
    d20, d21 = _edge_tiles(dn2_0, 16, n2, 64), _edge_tiles(dn2_1, 16, n2, 64)
    d10, d11 = _edge_tiles(dn1_0, 32, n1, 64), _edge_tiles(dn1_1, 32, n1, 64)
    rin, rout = _edge_tiles(ring_in, 8, NR), _edge_tiles(ring_out, 16, n1, 64)

    h1p, h2p = _pack2(h1), _pack2(h2)
    h3p, h4p = _pack2(h3), _pack2(h4)

    s2 = _seg_row_split(
        _pku(h1p, n1), [[u20], [u21]], n2, 2,
        chunk_of=lambda c, p: c, out_of=lambda c, p: p, n_out=2)
    s2 = s2.reshape(2, -1, 128)
    h2u = _mlp([(h2p, 'packed', Wup2_1[0:64]),
                (s2, ('slotp', 0, 2), Wup2_1[64:128]),
                (s2, ('slotp', 1, 2), Wup2_1[128:192])],
               bup2_1, Wup2_2, bup2_2, n2)

    s3 = _seg_etype_split(_pku(h2u, n2), [u30, u31], n3).reshape(2, -1, 128)
    h3u = _mlp([(h3p, 'packed', Wup3_1[0:64]),
                (s3, ('slotp', 0, 2), Wup3_1[64:128]),
                (s3, ('slotp', 1, 2), Wup3_1[128:192])],
               bup3_1, Wup3_2, bup3_2, n3)

    s4 = _seg_etype_split(_pku(h3u, n3), [u40, u41], n4).reshape(2, -1, 128)
    h4f = _mlp([(h4p, 'packed', Wup4_1[0:64]),
                (s4, ('slotp', 0, 2), Wup4_1[64:128]),
                (s4, ('slotp', 1, 2), Wup4_1[128:192])],
               bup4_1, Wup4_2, bup4_2, n4)

    d3 = _seg_edge_split(_pku(h4f, n4), [d30, d31], n3).reshape(NC, -1, 128)
    h3f = _mlp([(h3u, 'packed', Wdn3_1[0:64]),
                (d3, ('pairp',), Wdn3_1[64:128])],
               bdn3_1, Wdn3_2, bdn3_2, n3)

    d2 = _seg_row_split(
        _pku(h3f, n3), [[d20, d21]], n2, 2,
        chunk_of=lambda c, p: c, out_of=lambda c, p: 0, n_out=1)
    d2 = d2.reshape(1, -1, 128)
    h2f = _mlp([(h2u, 'packed', Wdn2_1[0:64]),
                (d2, ('slotp', 0, 1), Wdn2_1[64:128])],
               bdn2_1, Wdn2_2, bdn2_2, n2)

    d1 = _seg_row_split(
        _pku(h2f, n2), [[d10, d11], [d10, d11]], n1, 4,
        chunk_of=lambda c, p: 2 * c + p, out_of=lambda c, p: 0, n_out=1)
    d1 = d1.reshape(1, -1, 128)
    h1r = _mlp([(h1p, 'packed', Wdn1_1[0:64]),
                (d1, ('slotp', 0, 1), Wdn1_1[64:128])],
               bdn1_1, Wdn1_2, bdn1_2, n1)

    hrp = _seg_edge_split(_pku(h1r, n1), [rin], NR).reshape(NC, -1, 128)
    hr = _pair_sum(hrp, NR)
    r1 = _seg_row_split(
        _pku(hr, NR), [[rout], [rout]], n1, 4,
        chunk_of=lambda c, p: 2 * c + p, out_of=lambda c, p: 0, n_out=1)
    r1 = r1.reshape(1, -1, 128)
    h1f = _mlp([(h1r, 'packed', Wr_1[0:64]),
                (r1, ('slotp', 0, 1), Wr_1[64:128])],
               br_1, Wr_2, br_2, n1)

    return (_pku(h1f, n1), _pku(h2f, n2), _pku(h3f, n3), _pku(h4f, n4),
            _pku(hr, NR))


# speedup vs baseline: 3.3400x; 1.0026x over previous
import jax
import jax.numpy as jnp
from jax import lax
from jax.experimental import pallas as pl
from jax.experimental.pallas import tpu as pltpu
from jax.experimental.pallas import tpu_sc as plsc

U = 64
NC, NS = 2, 16
GB = 128
KPIPE = 6
BN = 512
ZR = 16
NDUMP = 64

_MESH = plsc.VectorSubcoreMesh(
    core_axis_name="c", subcore_axis_name="s", num_cores=NC, num_subcores=NS)
_SC_PARAMS = pltpu.CompilerParams(use_tc_tiling_on_sc=False)

_f32 = jnp.float32


def _acc_geom(n):
    rpw = -(-n // NS)
    rpw = -(-rpw // ZR) * ZR
    return rpw, NS * rpw


def _chunk_size(n, n_chunks):
    return -(-(-(-n // n_chunks)) // (NS * ZR)) * (NS * ZR)


def _edge_tiles(e, nb, n_dst, gb=GB):
    E = e.shape[1]
    tile = nb * gb
    Ep = -(-E // tile) * tile
    src, dst = e[0], e[1]
    if Ep != E:
        src = jnp.concatenate([src, jnp.zeros((Ep - E,), e.dtype)])
        dst = jnp.concatenate([dst, jnp.full((Ep - E,), n_dst, e.dtype)])
    return src.reshape(-1, nb, gb), dst.reshape(-1, nb, gb)


def _fill_zero(zbuf):
    zr, cols = zbuf.shape
    z = jnp.zeros((16,), _f32)

    @pl.loop(0, zr)
    def _(r):
        @pl.loop(0, cols, step=16)
        def _(cc):
            zbuf[r, pl.ds(cc, 16)] = z


def _zero_rows(acc, zbuf, lo, nrows, sem):
    nz = nrows // ZR

    @pl.loop(0, nz)
    def _(i):
        pltpu.async_copy(zbuf, acc.at[pl.ds(lo + i * ZR, ZR)], sem)

    @pl.loop(0, nz)
    def _(i):
        pltpu.make_async_copy(zbuf, acc.at[pl.ds(lo, ZR)], sem).wait()


def _stream_tiles(table_ref, src_r, dst_r, acc, sbuf, dbuf, rowbufs, gsems,
                  ssems, start, step, clamp=None):
    t_tot, nb, gb = src_r.shape

    D = KPIPE // 2

    @pl.loop(start, t_tot, step=step)
    def _(t):
        pltpu.sync_copy(src_r.at[t], sbuf)
        pltpu.sync_copy(dst_r.at[t], dbuf)
        if clamp is not None:
            lo, cs = clamp

            @pl.loop(0, nb)
            def _(jv):
                for m in range(gb // 16):
                    v = dbuf[jv, pl.ds(16 * m, 16)]
                    inr = (v >= lo) & (v < lo + cs)
                    dump = cs + (v & (NDUMP - 1))
                    dbuf[jv, pl.ds(16 * m, 16)] = jnp.where(inr, v - lo, dump)
        for j in range(min(D, nb)):
            pltpu.async_copy(table_ref.at[sbuf.at[j]], rowbufs[j % KPIPE],
                             gsems[j % KPIPE])
        for j in range(nb):
            k = j % KPIPE
            if j + D < nb:
                kd = (j + D) % KPIPE
                jprev = j + D - KPIPE
                if jprev >= 0:
                    pltpu.make_async_copy(rowbufs[kd], acc.at[dbuf.at[jprev]],
                                          ssems[kd]).wait()
                pltpu.async_copy(table_ref.at[sbuf.at[j + D]], rowbufs[kd],
                                 gsems[kd])
            pltpu.make_async_copy(table_ref.at[sbuf.at[j]], rowbufs[k],
                                  gsems[k]).wait()
            pltpu.async_copy(rowbufs[k], acc.at[dbuf.at[j]], ssems[k],
                             add=True)
        for j in range(max(0, nb - KPIPE), nb):
            k = j % KPIPE
            pltpu.make_async_copy(rowbufs[k], acc.at[dbuf.at[j]],
                                  ssems[k]).wait()


def _sc_scratch(acc_rows, nb, gb):
    return [pltpu.VMEM_SHARED((acc_rows, U), _f32),
            pltpu.VMEM((nb, gb), jnp.int32),
            pltpu.VMEM((nb, gb), jnp.int32)] \
        + [pltpu.VMEM((gb, U), _f32) for _ in range(KPIPE)] \
        + [pltpu.VMEM((ZR, U), _f32)] \
        + [pltpu.SemaphoreType.DMA] * (2 * KPIPE + 1)


def _unpack_refs(refs, n_stream_args):
    stream_refs = refs[:n_stream_args]
    rest = refs[n_stream_args:]
    acc, sbuf, dbuf = rest[:3]
    rowbufs = rest[3:3 + KPIPE]
    zbuf = rest[3 + KPIPE]
    gsems = rest[4 + KPIPE:4 + 2 * KPIPE]
    ssems = rest[4 + 2 * KPIPE:4 + 3 * KPIPE]
    zsem = rest[4 + 3 * KPIPE]
    return stream_refs, acc, sbuf, dbuf, rowbufs, zbuf, gsems, ssems, zsem


def _seg_edge_split(table, streams, n):
    rpw, npad = _acc_geom(n)
    n_str = len(streams)
    nb = streams[0][0].shape[1]

    def body(*refs):
        (srefs, acc, sbuf, dbuf, rowbufs, zbuf, gsems, ssems,
         zsem) = _unpack_refs(refs[1:1 + 2 * n_str] + refs[2 + 2 * n_str:],
                              2 * n_str)
        table_ref = refs[0]
        out = refs[1 + 2 * n_str]
        c = lax.axis_index("c")
        s = lax.axis_index("s")
        gw = c * NS + s
        _fill_zero(zbuf)
        _zero_rows(acc, zbuf, s * rpw, rpw, zsem)
        plsc.subcore_barrier()
        for k in range(n_str):
            _stream_tiles(table_ref, srefs[2 * k], srefs[2 * k + 1],
                          acc, sbuf, dbuf, rowbufs, gsems, ssems, gw, NC * NS)
        plsc.subcore_barrier()
        pltpu.sync_copy(acc.at[pl.ds(s * rpw, rpw)],
                        out.at[c, pl.ds(s * rpw, rpw)])

    out_type = jax.ShapeDtypeStruct((NC, npad, U), _f32)
    fn = pl.kernel(body, out_type=out_type, mesh=_MESH,
                   scratch_types=_sc_scratch(npad + ZR, nb,
                                             streams[0][0].shape[2]),
                   compiler_params=_SC_PARAMS)
    args = [table]
    for (sr, dr) in streams:
        args += [sr, dr]
    return fn(*args)


def _seg_etype_split(table, streams, n):
    rpw, npad = _acc_geom(n)
    nb = streams[0][0].shape[1]

    def body(*refs):
        (srefs, acc, sbuf, dbuf, rowbufs, zbuf, gsems, ssems,
         zsem) = _unpack_refs(refs[1:5] + refs[6:], 4)
        table_ref = refs[0]
        out = refs[5]
        c = lax.axis_index("c")
        s = lax.axis_index("s")
        _fill_zero(zbuf)
        _zero_rows(acc, zbuf, s * rpw, rpw, zsem)
        plsc.subcore_barrier()
        for k in range(2):
            @pl.when(c == k)
            def _(k=k):
                _stream_tiles(table_ref, srefs[2 * k], srefs[2 * k + 1],
                              acc, sbuf, dbuf, rowbufs, gsems, ssems, s, NS)
        plsc.subcore_barrier()
        pltpu.sync_copy(acc.at[pl.ds(s * rpw, rpw)],
                        out.at[c, pl.ds(s * rpw, rpw)])

    out_type = jax.ShapeDtypeStruct((2, npad, U), _f32)
    fn = pl.kernel(body, out_type=out_type, mesh=_MESH,
                   scratch_types=_sc_scratch(npad + ZR, nb,
                                             streams[0][0].shape[2]),
                   compiler_params=_SC_PARAMS)
    return fn(table, streams[0][0], streams[0][1], streams[1][0], streams[1][1])


def _seg_row_split(table, passes, n, n_chunks, chunk_of, out_of, n_out):
    CS = _chunk_size(n, n_chunks)
    rpw_c = CS // NS
    n_pass = len(passes)
    flat = [st for ps in passes for st in ps]
    nb = flat[0][0].shape[1]

    def body(*refs):
        (srefs, acc, sbuf, dbuf, rowbufs, zbuf, gsems, ssems,
         zsem) = _unpack_refs(refs[1:1 + 2 * len(flat)]
                              + refs[2 + 2 * len(flat):], 2 * len(flat))
        table_ref = refs[0]
        out = refs[1 + 2 * len(flat)]
        c = lax.axis_index("c")
        s = lax.axis_index("s")
        _fill_zero(zbuf)
        off = 0
        for p in range(n_pass):
            _zero_rows(acc, zbuf, s * rpw_c, rpw_c, zsem)
            plsc.subcore_barrier()
            q = chunk_of(c, p)
            oidx = out_of(c, p)
            lo = q * CS
            for k in range(len(passes[p])):
                _stream_tiles(table_ref, srefs[2 * (off + k)],
                              srefs[2 * (off + k) + 1],
                              acc, sbuf, dbuf, rowbufs, gsems, ssems,
                              s, NS, clamp=(lo, CS))
            off += len(passes[p])
            plsc.subcore_barrier()
            pltpu.sync_copy(acc.at[pl.ds(s * rpw_c, rpw_c)],
                            out.at[oidx, pl.ds(lo + s * rpw_c, rpw_c)])

    out_type = jax.ShapeDtypeStruct((n_out, n_chunks * CS, U), _f32)
    fn = pl.kernel(body, out_type=out_type, mesh=_MESH,
                   scratch_types=_sc_scratch(CS + NDUMP, nb,
                                             flat[0][0].shape[2]),
                   compiler_params=_SC_PARAMS)
    args = [table]
    for (sr, dr) in flat:
        args += [sr, dr]
    return fn(*args)


def _bd(W):
    k = W.shape[0]
    z = jnp.zeros((k, U), _f32)
    return jnp.concatenate(
        [jnp.concatenate([W, z], axis=1), jnp.concatenate([z, W], axis=1)],
        axis=0)


_bf16 = jnp.bfloat16


def _wsplit(W):
    Wh = W.astype(_bf16)
    Wl = (W - Wh.astype(_f32)).astype(_bf16)
    return Wh, Wl


def _dot3(x, Wh, Wl):
    xh = x.astype(_bf16)
    xl = (x - xh.astype(_f32)).astype(_bf16)
    return (jnp.dot(xh, Wh, preferred_element_type=_f32)
            + jnp.dot(xh, Wl, preferred_element_type=_f32)
            + jnp.dot(xl, Wh, preferred_element_type=_f32))


def _b2x(b):
    return jnp.concatenate([b, b]).reshape(1, 2 * U)


def _mlp(terms, b1, W2, b2, n):
    n_t = len(terms)

    def body(*refs):
        t_refs = refs[:n_t]
        w1_refs = refs[n_t:3 * n_t]
        b1_ref, w2h_ref, w2l_ref, b2_ref = refs[3 * n_t:3 * n_t + 4]
        o_ref = refs[3 * n_t + 4]
        acc = jnp.broadcast_to(b1_ref[...], (BN // 2, 2 * U)).astype(_f32)
        for t, (arr, kind, _) in enumerate(terms):
            if kind == 'packed':
                x = t_refs[t][...]
            elif kind[0] == 'pairp':
                x = t_refs[t][0] + t_refs[t][1]
            else:
                x = t_refs[t][0]
            acc = acc + _dot3(x, w1_refs[2 * t][...], w1_refs[2 * t + 1][...])
        h = jnp.tanh(acc)
        o_ref[...] = _dot3(h, w2h_ref[...], w2l_ref[...]) + b2_ref[...]

    in_specs = []
    args = []
    for (arr, kind, _) in terms:
        if kind == 'packed':
            in_specs.append(pl.BlockSpec((BN // 2, 128), lambda i: (i, 0)))
        elif kind[0] == 'pairp':
            in_specs.append(pl.BlockSpec((NC, BN // 2, 128),
                                         lambda i: (0, i, 0)))
        else:
            _, q, qs = kind
            in_specs.append(pl.BlockSpec((1, BN // 2, 128),
                                         lambda i, q=q: (q, i, 0)))
        args.append(arr)
    for (_, _, w1s) in terms:
        wh, wl = _wsplit(_bd(w1s))
        in_specs += [pl.BlockSpec((128, 128), lambda i: (0, 0))] * 2
        args += [wh, wl]
    w2h, w2l = _wsplit(_bd(W2))
    in_specs += [pl.BlockSpec((1, 128), lambda i: (0, 0)),
                 pl.BlockSpec((128, 128), lambda i: (0, 0)),
                 pl.BlockSpec((128, 128), lambda i: (0, 0)),
                 pl.BlockSpec((1, 128), lambda i: (0, 0))]
    args += [_b2x(b1), w2h, w2l, _b2x(b2)]
    return pl.pallas_call(
        body, grid=(pl.cdiv(n, BN),),
        in_specs=in_specs,
        out_specs=pl.BlockSpec((BN // 2, 128), lambda i: (i, 0)),
        out_shape=jax.ShapeDtypeStruct((n * U // 128, 128), _f32))(*args)


def _pair_sum(parts, n):
    def body(p_ref, o_ref):
        o_ref[...] = p_ref[0] + p_ref[1]

    return pl.pallas_call(
        body, grid=(pl.cdiv(n, BN),),
        in_specs=[pl.BlockSpec((NC, BN // 2, 128), lambda i: (0, i, 0))],
        out_specs=pl.BlockSpec((BN // 2, 128), lambda i: (i, 0)),
        out_shape=jax.ShapeDtypeStruct((n * U // 128, 128), _f32))(parts)


def _pack2(x):
    n = x.shape[0]
    return lax.optimization_barrier(x.reshape(n * U // 128, 128))


def _pku(x, n):
    return x.reshape(-1, U)[:n]


def kernel(h1, h2, h3, h4, Wup2_1, bup2_1, Wup2_2, bup2_2, Wup3_1, bup3_1, Wup3_2, bup3_2, Wup4_1, bup4_1, Wup4_2, bup4_2, Wdn1_1, bdn1_1, Wdn1_2, bdn1_2, Wdn2_1, bdn2_1, Wdn2_2, bdn2_2, Wdn3_1, bdn3_1, Wdn3_2, bdn3_2, Wr_1, br_1, Wr_2, br_2, up2_0, up2_1, up3_0, up3_1, up4_0, up4_1, dn3_0, dn3_1, dn2_0, dn2_1, dn1_0, dn1_1, ring_in, ring_out):
    n1, n2, n3, n4 = h1.shape[0], h2.shape[0], h3.shape[0], h4.shape[0]
    NR = 10000

    u20, u21 = _edge_tiles(up2_0, 32, n2, 64), _edge_tiles(up2_1, 32, n2, 64)
    u30, u31 = _edge_tiles(up3_0, 8, n3), _edge_tiles(up3_1, 8, n3)
    u40, u41 = _edge_tiles(up4_0, 4, n4), _edge_tiles(up4_1, 4, n4)
    d30, d31 = _edge_tiles(dn3_0, 4, n3), _edge_tiles(dn3_1, 4, n3)
    d20, d21 = _edge_tiles(dn2_0, 16, n2, 64), _edge_tiles(dn2_1, 16, n2, 64)
    d10, d11 = _edge_tiles(dn1_0, 32, n1, 64), _edge_tiles(dn1_1, 32, n1, 64)
    rin, rout = _edge_tiles(ring_in, 8, NR), _edge_tiles(ring_out, 16, n1, 64)

    h1p, h2p = _pack2(h1), _pack2(h2)
    h3p, h4p = _pack2(h3), _pack2(h4)

    s2 = _seg_row_split(
        _pku(h1p, n1), [[u20], [u21]], n2, 2,
        chunk_of=lambda c, p: c, out_of=lambda c, p: p, n_out=2)
    s2 = s2.reshape(2, -1, 128)
    h2u = _mlp([(h2p, 'packed', Wup2_1[0:64]),
                (s2, ('slotp', 0, 2), Wup2_1[64:128]),
                (s2, ('slotp', 1, 2), Wup2_1[128:192])],
               bup2_1, Wup2_2, bup2_2, n2)

    s3 = _seg_etype_split(_pku(h2u, n2), [u30, u31], n3).reshape(2, -1, 128)
    h3u = _mlp([(h3p, 'packed', Wup3_1[0:64]),
                (s3, ('slotp', 0, 2), Wup3_1[64:128]),
                (s3, ('slotp', 1, 2), Wup3_1[128:192])],
               bup3_1, Wup3_2, bup3_2, n3)

    s4 = _seg_etype_split(_pku(h3u, n3), [u40, u41], n4).reshape(2, -1, 128)
    h4f = _mlp([(h4p, 'packed', Wup4_1[0:64]),
                (s4, ('slotp', 0, 2), Wup4_1[64:128]),
                (s4, ('slotp', 1, 2), Wup4_1[128:192])],
               bup4_1, Wup4_2, bup4_2, n4)

    d3 = _seg_edge_split(_pku(h4f, n4), [d30, d31], n3).reshape(NC, -1, 128)
    h3f = _mlp([(h3u, 'packed', Wdn3_1[0:64]),
                (d3, ('pairp',), Wdn3_1[64:128])],
               bdn3_1, Wdn3_2, bdn3_2, n3)

    d2 = _seg_row_split(
        _pku(h3f, n3), [[d20, d21]], n2, 2,
        chunk_of=lambda c, p: c, out_of=lambda c, p: 0, n_out=1)
    d2 = d2.reshape(1, -1, 128)
    h2f = _mlp([(h2u, 'packed', Wdn2_1[0:64]),
                (d2, ('slotp', 0, 1), Wdn2_1[64:128])],
               bdn2_1, Wdn2_2, bdn2_2, n2)

    d1 = _seg_row_split(
        _pku(h2f, n2), [[d10, d11], [d10, d11]], n1, 4,
        chunk_of=lambda c, p: 2 * c + p, out_of=lambda c, p: 0, n_out=1)
    d1 = d1.reshape(1, -1, 128)
    h1r = _mlp([(h1p, 'packed', Wdn1_1[0:64]),
                (d1, ('slotp', 0, 1), Wdn1_1[64:128])],
               bdn1_1, Wdn1_2, bdn1_2, n1)

    hrp = _seg_edge_split(_pku(h1r, n1), [rin], NR).reshape(NC, -1, 128)
    hr = _pair_sum(hrp, NR)
    r1 = _seg_row_split(
        _pku(hr, NR), [[rout], [rout]], n1, 4,
        chunk_of=lambda c, p: 2 * c + p, out_of=lambda c, p: 0, n_out=1)
    r1 = r1.reshape(1, -1, 128)
    h1f = _mlp([(h1r, 'packed', Wr_1[0:64]),
                (r1, ('slotp', 0, 1), Wr_1[64:128])],
               br_1, Wr_2, br_2, n1)

    return (_pku(h1f, n1), _pku(h2f, n2), _pku(h3f, n3), _pku(h4f, n4),
            _pku(hr, NR))


# speedup vs baseline: 4.2958x; 1.2862x over previous
import jax
import jax.numpy as jnp
from jax import lax
from jax.experimental import pallas as pl
from jax.experimental.pallas import tpu as pltpu
from jax.experimental.pallas import tpu_sc as plsc

U = 64
NC, NS = 2, 16
GB = 128
KPIPE = 6
BN = 2048
ZR = 16
NDUMP = 64

_MESH = plsc.VectorSubcoreMesh(
    core_axis_name="c", subcore_axis_name="s", num_cores=NC, num_subcores=NS)
_SC_PARAMS = pltpu.CompilerParams(use_tc_tiling_on_sc=False)

_f32 = jnp.float32


def _acc_geom(n):
    rpw = -(-n // NS)
    rpw = -(-rpw // ZR) * ZR
    return rpw, NS * rpw


def _chunk_size(n, n_chunks):
    return -(-(-(-n // n_chunks)) // (NS * ZR)) * (NS * ZR)


def _edge_tiles(e, nb, n_dst, gb=GB):
    E = e.shape[1]
    tile = nb * gb
    Ep = -(-E // tile) * tile
    src, dst = e[0], e[1]
    if Ep != E:
        src = jnp.concatenate([src, jnp.zeros((Ep - E,), e.dtype)])
        dst = jnp.concatenate([dst, jnp.full((Ep - E,), n_dst, e.dtype)])
    return src.reshape(-1, nb, gb), dst.reshape(-1, nb, gb)


def _fill_zero(zbuf):
    zr, cols = zbuf.shape
    z = jnp.zeros((16,), _f32)

    @pl.loop(0, zr)
    def _(r):
        @pl.loop(0, cols, step=16)
        def _(cc):
            zbuf[r, pl.ds(cc, 16)] = z


def _zero_rows(acc, zbuf, lo, nrows, sem):
    nz = nrows // ZR

    @pl.loop(0, nz)
    def _(i):
        pltpu.async_copy(zbuf, acc.at[pl.ds(lo + i * ZR, ZR)], sem)

    @pl.loop(0, nz)
    def _(i):
        pltpu.make_async_copy(zbuf, acc.at[pl.ds(lo, ZR)], sem).wait()


def _stream_tiles(table_ref, src_r, dst_r, acc, sbuf, dbuf, rowbufs, gsems,
                  ssems, start, step, clamp=None):
    t_tot, nb, gb = src_r.shape

    D = KPIPE // 2

    @pl.loop(start, t_tot, step=step)
    def _(t):
        pltpu.sync_copy(src_r.at[t], sbuf)
        pltpu.sync_copy(dst_r.at[t], dbuf)
        if clamp is not None:
            lo, cs = clamp

            @pl.loop(0, nb)
            def _(jv):
                for m in range(gb // 16):
                    v = dbuf[jv, pl.ds(16 * m, 16)]
                    inr = (v >= lo) & (v < lo + cs)
                    dump = cs + (v & (NDUMP - 1))
                    dbuf[jv, pl.ds(16 * m, 16)] = jnp.where(inr, v - lo, dump)
        for j in range(min(D, nb)):
            pltpu.async_copy(table_ref.at[sbuf.at[j]], rowbufs[j % KPIPE],
                             gsems[j % KPIPE])
        for j in range(nb):
            k = j % KPIPE
            if j + D < nb:
                kd = (j + D) % KPIPE
                jprev = j + D - KPIPE
                if jprev >= 0:
                    pltpu.make_async_copy(rowbufs[kd], acc.at[dbuf.at[jprev]],
                                          ssems[kd]).wait()
                pltpu.async_copy(table_ref.at[sbuf.at[j + D]], rowbufs[kd],
                                 gsems[kd])
            pltpu.make_async_copy(table_ref.at[sbuf.at[j]], rowbufs[k],
                                  gsems[k]).wait()
            pltpu.async_copy(rowbufs[k], acc.at[dbuf.at[j]], ssems[k],
                             add=True)
        for j in range(max(0, nb - KPIPE), nb):
            k = j % KPIPE
            pltpu.make_async_copy(rowbufs[k], acc.at[dbuf.at[j]],
                                  ssems[k]).wait()


def _sc_scratch(acc_rows, nb, gb):
    return [pltpu.VMEM_SHARED((acc_rows, U), _f32),
            pltpu.VMEM((nb, gb), jnp.int32),
            pltpu.VMEM((nb, gb), jnp.int32)] \
        + [pltpu.VMEM((gb, U), _f32) for _ in range(KPIPE)] \
        + [pltpu.VMEM((ZR, U), _f32)] \
        + [pltpu.SemaphoreType.DMA] * (2 * KPIPE + 1)


def _unpack_refs(refs, n_stream_args):
    stream_refs = refs[:n_stream_args]
    rest = refs[n_stream_args:]
    acc, sbuf, dbuf = rest[:3]
    rowbufs = rest[3:3 + KPIPE]
    zbuf = rest[3 + KPIPE]
    gsems = rest[4 + KPIPE:4 + 2 * KPIPE]
    ssems = rest[4 + 2 * KPIPE:4 + 3 * KPIPE]
    zsem = rest[4 + 3 * KPIPE]
    return stream_refs, acc, sbuf, dbuf, rowbufs, zbuf, gsems, ssems, zsem


def _seg_edge_split(table, streams, n):
    rpw, npad = _acc_geom(n)
    n_str = len(streams)
    nb = streams[0][0].shape[1]

    def body(*refs):
        (srefs, acc, sbuf, dbuf, rowbufs, zbuf, gsems, ssems,
         zsem) = _unpack_refs(refs[1:1 + 2 * n_str] + refs[2 + 2 * n_str:],
                              2 * n_str)
        table_ref = refs[0]
        out = refs[1 + 2 * n_str]
        c = lax.axis_index("c")
        s = lax.axis_index("s")
        gw = c * NS + s
        _fill_zero(zbuf)
        _zero_rows(acc, zbuf, s * rpw, rpw, zsem)
        plsc.subcore_barrier()
        for k in range(n_str):
            _stream_tiles(table_ref, srefs[2 * k], srefs[2 * k + 1],
                          acc, sbuf, dbuf, rowbufs, gsems, ssems, gw, NC * NS)
        plsc.subcore_barrier()
        pltpu.sync_copy(acc.at[pl.ds(s * rpw, rpw)],
                        out.at[c, pl.ds(s * rpw, rpw)])

    out_type = jax.ShapeDtypeStruct((NC, npad, U), _f32)
    fn = pl.kernel(body, out_type=out_type, mesh=_MESH,
                   scratch_types=_sc_scratch(npad + ZR, nb,
                                             streams[0][0].shape[2]),
                   compiler_params=_SC_PARAMS)
    args = [table]
    for (sr, dr) in streams:
        args += [sr, dr]
    return fn(*args)


def _seg_etype_split(table, streams, n):
    rpw, npad = _acc_geom(n)
    nb = streams[0][0].shape[1]

    def body(*refs):
        (srefs, acc, sbuf, dbuf, rowbufs, zbuf, gsems, ssems,
         zsem) = _unpack_refs(refs[1:5] + refs[6:], 4)
        table_ref = refs[0]
        out = refs[5]
        c = lax.axis_index("c")
        s = lax.axis_index("s")
        _fill_zero(zbuf)
        _zero_rows(acc, zbuf, s * rpw, rpw, zsem)
        plsc.subcore_barrier()
        for k in range(2):
            @pl.when(c == k)
            def _(k=k):
                _stream_tiles(table_ref, srefs[2 * k], srefs[2 * k + 1],
                              acc, sbuf, dbuf, rowbufs, gsems, ssems, s, NS)
        plsc.subcore_barrier()
        pltpu.sync_copy(acc.at[pl.ds(s * rpw, rpw)],
                        out.at[c, pl.ds(s * rpw, rpw)])

    out_type = jax.ShapeDtypeStruct((2, npad, U), _f32)
    fn = pl.kernel(body, out_type=out_type, mesh=_MESH,
                   scratch_types=_sc_scratch(npad + ZR, nb,
                                             streams[0][0].shape[2]),
                   compiler_params=_SC_PARAMS)
    return fn(table, streams[0][0], streams[0][1], streams[1][0], streams[1][1])


def _seg_row_split(table, passes, n, n_chunks, chunk_of, out_of, n_out):
    CS = _chunk_size(n, n_chunks)
    rpw_c = CS // NS
    n_pass = len(passes)
    flat = [st for ps in passes for st in ps]
    nb = flat[0][0].shape[1]

    def body(*refs):
        (srefs, acc, sbuf, dbuf, rowbufs, zbuf, gsems, ssems,
         zsem) = _unpack_refs(refs[1:1 + 2 * len(flat)]
                              + refs[2 + 2 * len(flat):], 2 * len(flat))
        table_ref = refs[0]
        out = refs[1 + 2 * len(flat)]
        c = lax.axis_index("c")
        s = lax.axis_index("s")
        _fill_zero(zbuf)
        off = 0
        for p in range(n_pass):
            _zero_rows(acc, zbuf, s * rpw_c, rpw_c, zsem)
            plsc.subcore_barrier()
            q = chunk_of(c, p)
            oidx = out_of(c, p)
            lo = q * CS
            for k in range(len(passes[p])):
                _stream_tiles(table_ref, srefs[2 * (off + k)],
                              srefs[2 * (off + k) + 1],
                              acc, sbuf, dbuf, rowbufs, gsems, ssems,
                              s, NS, clamp=(lo, CS))
            off += len(passes[p])
            plsc.subcore_barrier()
            pltpu.sync_copy(acc.at[pl.ds(s * rpw_c, rpw_c)],
                            out.at[oidx, pl.ds(lo + s * rpw_c, rpw_c)])

    out_type = jax.ShapeDtypeStruct((n_out, n_chunks * CS, U), _f32)
    fn = pl.kernel(body, out_type=out_type, mesh=_MESH,
                   scratch_types=_sc_scratch(CS + NDUMP, nb,
                                             flat[0][0].shape[2]),
                   compiler_params=_SC_PARAMS)
    args = [table]
    for (sr, dr) in flat:
        args += [sr, dr]
    return fn(*args)


def _bd(W):
    k = W.shape[0]
    z = jnp.zeros((k, U), _f32)
    return jnp.concatenate(
        [jnp.concatenate([W, z], axis=1), jnp.concatenate([z, W], axis=1)],
        axis=0)


_bf16 = jnp.bfloat16


def _wsplit(W):
    Wh = W.astype(_bf16)
    Wl = (W - Wh.astype(_f32)).astype(_bf16)
    return Wh, Wl


def _dot3(x, Wh, Wl):
    xh = x.astype(_bf16)
    xl = (x - xh.astype(_f32)).astype(_bf16)
    return (jnp.dot(xh, Wh, preferred_element_type=_f32)
            + jnp.dot(xh, Wl, preferred_element_type=_f32)
            + jnp.dot(xl, Wh, preferred_element_type=_f32))


def _b2x(b):
    return jnp.concatenate([b, b]).reshape(1, 2 * U)


def _mlp(terms, b1, W2, b2, n):
    n_t = len(terms)

    def body(*refs):
        t_refs = refs[:n_t]
        w1_refs = refs[n_t:2 * n_t]
        b1_ref, w2_ref, b2_ref = refs[2 * n_t:2 * n_t + 3]
        o_ref = refs[2 * n_t + 3]
        acc = jnp.broadcast_to(b1_ref[...], (BN // 2, 2 * U)).astype(_f32)
        for t, (arr, kind, _) in enumerate(terms):
            if kind == 'packed':
                x = t_refs[t][...]
            elif kind[0] == 'pairp':
                x = t_refs[t][0] + t_refs[t][1]
            else:
                x = t_refs[t][0]
            acc = acc + jnp.dot(x, w1_refs[t][...], preferred_element_type=_f32)
        h = jnp.tanh(acc)
        o_ref[...] = jnp.dot(h, w2_ref[...],
                             preferred_element_type=_f32) + b2_ref[...]

    in_specs = []
    args = []
    for (arr, kind, _) in terms:
        if kind == 'packed':
            in_specs.append(pl.BlockSpec((BN // 2, 128), lambda i: (i, 0)))
        elif kind[0] == 'pairp':
            in_specs.append(pl.BlockSpec((NC, BN // 2, 128),
                                         lambda i: (0, i, 0)))
        else:
            _, q, qs = kind
            in_specs.append(pl.BlockSpec((1, BN // 2, 128),
                                         lambda i, q=q: (q, i, 0)))
        args.append(arr)
    for (_, _, w1s) in terms:
        in_specs.append(pl.BlockSpec((128, 128), lambda i: (0, 0)))
        args.append(_bd(w1s))
    in_specs += [pl.BlockSpec((1, 128), lambda i: (0, 0)),
                 pl.BlockSpec((128, 128), lambda i: (0, 0)),
                 pl.BlockSpec((1, 128), lambda i: (0, 0))]
    args += [_b2x(b1), _bd(W2), _b2x(b2)]
    return pl.pallas_call(
        body, grid=(pl.cdiv(n, BN),),
        in_specs=in_specs,
        out_specs=pl.BlockSpec((BN // 2, 128), lambda i: (i, 0)),
        out_shape=jax.ShapeDtypeStruct((n * U // 128, 128), _f32))(*args)


def _pair_sum(parts, n):
    def body(p_ref, o_ref):
        o_ref[...] = p_ref[0] + p_ref[1]

    return pl.pallas_call(
        body, grid=(pl.cdiv(n, BN),),
        in_specs=[pl.BlockSpec((NC, BN // 2, 128), lambda i: (0, i, 0))],
        out_specs=pl.BlockSpec((BN // 2, 128), lambda i: (i, 0)),
        out_shape=jax.ShapeDtypeStruct((n * U // 128, 128), _f32))(parts)


def _pack2(x):
    n = x.shape[0]
    return lax.optimization_barrier(x.reshape(n * U // 128, 128))


def _pku(x, n):
    return x.reshape(-1, U)[:n]


def kernel(h1, h2, h3, h4, Wup2_1, bup2_1, Wup2_2, bup2_2, Wup3_1, bup3_1, Wup3_2, bup3_2, Wup4_1, bup4_1, Wup4_2, bup4_2, Wdn1_1, bdn1_1, Wdn1_2, bdn1_2, Wdn2_1, bdn2_1, Wdn2_2, bdn2_2, Wdn3_1, bdn3_1, Wdn3_2, bdn3_2, Wr_1, br_1, Wr_2, br_2, up2_0, up2_1, up3_0, up3_1, up4_0, up4_1, dn3_0, dn3_1, dn2_0, dn2_1, dn1_0, dn1_1, ring_in, ring_out):
    n1, n2, n3, n4 = h1.shape[0], h2.shape[0], h3.shape[0], h4.shape[0]
    NR = 10000

    u20, u21 = _edge_tiles(up2_0, 32, n2, 64), _edge_tiles(up2_1, 32, n2, 64)
    u30, u31 = _edge_tiles(up3_0, 8, n3), _edge_tiles(up3_1, 8, n3)
    u40, u41 = _edge_tiles(up4_0, 4, n4), _edge_tiles(up4_1, 4, n4)
    d30, d31 = _edge_tiles(dn3_0, 4, n3), _edge_tiles(dn3_1, 4, n3)
    d20, d21 = _edge_tiles(dn2_0, 16, n2, 64), _edge_tiles(dn2_1, 16, n2, 64)
    d10, d11 = _edge_tiles(dn1_0, 32, n1, 64), _edge_tiles(dn1_1, 32, n1, 64)
    rin, rout = _edge_tiles(ring_in, 8, NR), _edge_tiles(ring_out, 16, n1, 64)

    h1p, h2p = _pack2(h1), _pack2(h2)
    h3p, h4p = _pack2(h3), _pack2(h4)

    s2 = _seg_row_split(
        _pku(h1p, n1), [[u20], [u21]], n2, 2,
        chunk_of=lambda c, p: c, out_of=lambda c, p: p, n_out=2)
    s2 = s2.reshape(2, -1, 128)
    h2u = _mlp([(h2p, 'packed', Wup2_1[0:64]),
                (s2, ('slotp', 0, 2), Wup2_1[64:128]),
                (s2, ('slotp', 1, 2), Wup2_1[128:192])],
               bup2_1, Wup2_2, bup2_2, n2)

    s3 = _seg_etype_split(_pku(h2u, n2), [u30, u31], n3).reshape(2, -1, 128)
    h3u = _mlp([(h3p, 'packed', Wup3_1[0:64]),
                (s3, ('slotp', 0, 2), Wup3_1[64:128]),
                (s3, ('slotp', 1, 2), Wup3_1[128:192])],
               bup3_1, Wup3_2, bup3_2, n3)

    s4 = _seg_etype_split(_pku(h3u, n3), [u40, u41], n4).reshape(2, -1, 128)
    h4f = _mlp([(h4p, 'packed', Wup4_1[0:64]),
                (s4, ('slotp', 0, 2), Wup4_1[64:128]),
                (s4, ('slotp', 1, 2), Wup4_1[128:192])],
               bup4_1, Wup4_2, bup4_2, n4)

    d3 = _seg_edge_split(_pku(h4f, n4), [d30, d31], n3).reshape(NC, -1, 128)
    h3f = _mlp([(h3u, 'packed', Wdn3_1[0:64]),
                (d3, ('pairp',), Wdn3_1[64:128])],
               bdn3_1, Wdn3_2, bdn3_2, n3)

    d2 = _seg_row_split(
        _pku(h3f, n3), [[d20, d21]], n2, 2,
        chunk_of=lambda c, p: c, out_of=lambda c, p: 0, n_out=1)
    d2 = d2.reshape(1, -1, 128)
    h2f = _mlp([(h2u, 'packed', Wdn2_1[0:64]),
                (d2, ('slotp', 0, 1), Wdn2_1[64:128])],
               bdn2_1, Wdn2_2, bdn2_2, n2)

    d1 = _seg_row_split(
        _pku(h2f, n2), [[d10, d11], [d10, d11]], n1, 4,
        chunk_of=lambda c, p: 2 * c + p, out_of=lambda c, p: 0, n_out=1)
    d1 = d1.reshape(1, -1, 128)
    h1r = _mlp([(h1p, 'packed', Wdn1_1[0:64]),
                (d1, ('slotp', 0, 1), Wdn1_1[64:128])],
               bdn1_1, Wdn1_2, bdn1_2, n1)

    hrp = _seg_edge_split(_pku(h1r, n1), [rin], NR).reshape(NC, -1, 128)
    hr = _pair_sum(hrp, NR)
    r1 = _seg_row_split(
        _pku(hr, NR), [[rout], [rout]], n1, 4,
        chunk_of=lambda c, p: 2 * c + p, out_of=lambda c, p: 0, n_out=1)
    r1 = r1.reshape(1, -1, 128)
    h1f = _mlp([(h1r, 'packed', Wr_1[0:64]),
                (r1, ('slotp', 0, 1), Wr_1[64:128])],
               br_1, Wr_2, br_2, n1)

    return (_pku(h1f, n1), _pku(h2f, n2), _pku(h3f, n3), _pku(h4f, n4),
            _pku(hr, NR))


# speedup vs baseline: 4.4573x; 1.0376x over previous
import jax
import jax.numpy as jnp
from jax import lax
from jax.experimental import pallas as pl
from jax.experimental.pallas import tpu as pltpu
from jax.experimental.pallas import tpu_sc as plsc

U = 64
NC, NS = 2, 16
GB = 128
KPIPE = 6
BN = 4096
ZR = 16
NDUMP = 64

_MESH = plsc.VectorSubcoreMesh(
    core_axis_name="c", subcore_axis_name="s", num_cores=NC, num_subcores=NS)
_SC_PARAMS = pltpu.CompilerParams(use_tc_tiling_on_sc=False)

_f32 = jnp.float32


def _acc_geom(n):
    rpw = -(-n // NS)
    rpw = -(-rpw // ZR) * ZR
    return rpw, NS * rpw


def _chunk_size(n, n_chunks):
    return -(-(-(-n // n_chunks)) // (NS * ZR)) * (NS * ZR)


def _edge_tiles(e, nb, n_dst, gb=GB):
    E = e.shape[1]
    tile = nb * gb
    Ep = -(-E // tile) * tile
    src, dst = e[0], e[1]
    if Ep != E:
        src = jnp.concatenate([src, jnp.zeros((Ep - E,), e.dtype)])
        dst = jnp.concatenate([dst, jnp.full((Ep - E,), n_dst, e.dtype)])
    return src.reshape(-1, nb, gb), dst.reshape(-1, nb, gb)


def _fill_zero(zbuf):
    zr, cols = zbuf.shape
    z = jnp.zeros((16,), _f32)

    @pl.loop(0, zr)
    def _(r):
        @pl.loop(0, cols, step=16)
        def _(cc):
            zbuf[r, pl.ds(cc, 16)] = z


def _zero_rows(acc, zbuf, lo, nrows, sem):
    nz = nrows // ZR

    @pl.loop(0, nz)
    def _(i):
        pltpu.async_copy(zbuf, acc.at[pl.ds(lo + i * ZR, ZR)], sem)

    @pl.loop(0, nz)
    def _(i):
        pltpu.make_async_copy(zbuf, acc.at[pl.ds(lo, ZR)], sem).wait()


def _stream_tiles(table_ref, src_r, dst_r, acc, sbuf, dbuf, rowbufs, gsems,
                  ssems, start, step, clamp=None):
    t_tot, nb, gb = src_r.shape

    D = KPIPE // 2

    @pl.loop(start, t_tot, step=step)
    def _(t):
        pltpu.sync_copy(src_r.at[t], sbuf)
        pltpu.sync_copy(dst_r.at[t], dbuf)
        if clamp is not None:
            lo, cs = clamp

            @pl.loop(0, nb)
            def _(jv):
                for m in range(gb // 16):
                    v = dbuf[jv, pl.ds(16 * m, 16)]
                    inr = (v >= lo) & (v < lo + cs)
                    dump = cs + (v & (NDUMP - 1))
                    dbuf[jv, pl.ds(16 * m, 16)] = jnp.where(inr, v - lo, dump)
        for j in range(min(D, nb)):
            pltpu.async_copy(table_ref.at[sbuf.at[j]], rowbufs[j % KPIPE],
                             gsems[j % KPIPE])
        for j in range(nb):
            k = j % KPIPE
            if j + D < nb:
                kd = (j + D) % KPIPE
                jprev = j + D - KPIPE
                if jprev >= 0:
                    pltpu.make_async_copy(rowbufs[kd], acc.at[dbuf.at[jprev]],
                                          ssems[kd]).wait()
                pltpu.async_copy(table_ref.at[sbuf.at[j + D]], rowbufs[kd],
                                 gsems[kd])
            pltpu.make_async_copy(table_ref.at[sbuf.at[j]], rowbufs[k],
                                  gsems[k]).wait()
            pltpu.async_copy(rowbufs[k], acc.at[dbuf.at[j]], ssems[k],
                             add=True)
        for j in range(max(0, nb - KPIPE), nb):
            k = j % KPIPE
            pltpu.make_async_copy(rowbufs[k], acc.at[dbuf.at[j]],
                                  ssems[k]).wait()


def _sc_scratch(acc_rows, nb, gb):
    return [pltpu.VMEM_SHARED((acc_rows, U), _f32),
            pltpu.VMEM((nb, gb), jnp.int32),
            pltpu.VMEM((nb, gb), jnp.int32)] \
        + [pltpu.VMEM((gb, U), _f32) for _ in range(KPIPE)] \
        + [pltpu.VMEM((ZR, U), _f32)] \
        + [pltpu.SemaphoreType.DMA] * (2 * KPIPE + 1)


def _unpack_refs(refs, n_stream_args):
    stream_refs = refs[:n_stream_args]
    rest = refs[n_stream_args:]
    acc, sbuf, dbuf = rest[:3]
    rowbufs = rest[3:3 + KPIPE]
    zbuf = rest[3 + KPIPE]
    gsems = rest[4 + KPIPE:4 + 2 * KPIPE]
    ssems = rest[4 + 2 * KPIPE:4 + 3 * KPIPE]
    zsem = rest[4 + 3 * KPIPE]
    return stream_refs, acc, sbuf, dbuf, rowbufs, zbuf, gsems, ssems, zsem


def _seg_edge_split(table, streams, n):
    rpw, npad = _acc_geom(n)
    n_str = len(streams)
    nb = streams[0][0].shape[1]

    def body(*refs):
        (srefs, acc, sbuf, dbuf, rowbufs, zbuf, gsems, ssems,
         zsem) = _unpack_refs(refs[1:1 + 2 * n_str] + refs[2 + 2 * n_str:],
                              2 * n_str)
        table_ref = refs[0]
        out = refs[1 + 2 * n_str]
        c = lax.axis_index("c")
        s = lax.axis_index("s")
        gw = c * NS + s
        _fill_zero(zbuf)
        _zero_rows(acc, zbuf, s * rpw, rpw, zsem)
        plsc.subcore_barrier()
        for k in range(n_str):
            _stream_tiles(table_ref, srefs[2 * k], srefs[2 * k + 1],
                          acc, sbuf, dbuf, rowbufs, gsems, ssems, gw, NC * NS)
        plsc.subcore_barrier()
        pltpu.sync_copy(acc.at[pl.ds(s * rpw, rpw)],
                        out.at[c, pl.ds(s * rpw, rpw)])

    out_type = jax.ShapeDtypeStruct((NC, npad, U), _f32)
    fn = pl.kernel(body, out_type=out_type, mesh=_MESH,
                   scratch_types=_sc_scratch(npad + ZR, nb,
                                             streams[0][0].shape[2]),
                   compiler_params=_SC_PARAMS)
    args = [table]
    for (sr, dr) in streams:
        args += [sr, dr]
    return fn(*args)


def _seg_etype_split(table, streams, n):
    rpw, npad = _acc_geom(n)
    nb = streams[0][0].shape[1]

    def body(*refs):
        (srefs, acc, sbuf, dbuf, rowbufs, zbuf, gsems, ssems,
         zsem) = _unpack_refs(refs[1:5] + refs[6:], 4)
        table_ref = refs[0]
        out = refs[5]
        c = lax.axis_index("c")
        s = lax.axis_index("s")
        _fill_zero(zbuf)
        _zero_rows(acc, zbuf, s * rpw, rpw, zsem)
        plsc.subcore_barrier()
        for k in range(2):
            @pl.when(c == k)
            def _(k=k):
                _stream_tiles(table_ref, srefs[2 * k], srefs[2 * k + 1],
                              acc, sbuf, dbuf, rowbufs, gsems, ssems, s, NS)
        plsc.subcore_barrier()
        pltpu.sync_copy(acc.at[pl.ds(s * rpw, rpw)],
                        out.at[c, pl.ds(s * rpw, rpw)])

    out_type = jax.ShapeDtypeStruct((2, npad, U), _f32)
    fn = pl.kernel(body, out_type=out_type, mesh=_MESH,
                   scratch_types=_sc_scratch(npad + ZR, nb,
                                             streams[0][0].shape[2]),
                   compiler_params=_SC_PARAMS)
    return fn(table, streams[0][0], streams[0][1], streams[1][0], streams[1][1])


def _seg_row_split(table, passes, n, n_chunks, chunk_of, out_of, n_out):
    CS = _chunk_size(n, n_chunks)
    rpw_c = CS // NS
    n_pass = len(passes)
    flat = [st for ps in passes for st in ps]
    nb = flat[0][0].shape[1]

    def body(*refs):
        (srefs, acc, sbuf, dbuf, rowbufs, zbuf, gsems, ssems,
         zsem) = _unpack_refs(refs[1:1 + 2 * len(flat)]
                              + refs[2 + 2 * len(flat):], 2 * len(flat))
        table_ref = refs[0]
        out = refs[1 + 2 * len(flat)]
        c = lax.axis_index("c")
        s = lax.axis_index("s")
        _fill_zero(zbuf)
        off = 0
        for p in range(n_pass):
            _zero_rows(acc, zbuf, s * rpw_c, rpw_c, zsem)
            plsc.subcore_barrier()
            q = chunk_of(c, p)
            oidx = out_of(c, p)
            lo = q * CS
            for k in range(len(passes[p])):
                _stream_tiles(table_ref, srefs[2 * (off + k)],
                              srefs[2 * (off + k) + 1],
                              acc, sbuf, dbuf, rowbufs, gsems, ssems,
                              s, NS, clamp=(lo, CS))
            off += len(passes[p])
            plsc.subcore_barrier()
            pltpu.sync_copy(acc.at[pl.ds(s * rpw_c, rpw_c)],
                            out.at[oidx, pl.ds(lo + s * rpw_c, rpw_c)])

    out_type = jax.ShapeDtypeStruct((n_out, n_chunks * CS, U), _f32)
    fn = pl.kernel(body, out_type=out_type, mesh=_MESH,
                   scratch_types=_sc_scratch(CS + NDUMP, nb,
                                             flat[0][0].shape[2]),
                   compiler_params=_SC_PARAMS)
    args = [table]
    for (sr, dr) in flat:
        args += [sr, dr]
    return fn(*args)


def _bd(W):
    k = W.shape[0]
    z = jnp.zeros((k, U), _f32)
    return jnp.concatenate(
        [jnp.concatenate([W, z], axis=1), jnp.concatenate([z, W], axis=1)],
        axis=0)


_bf16 = jnp.bfloat16


def _wsplit(W):
    Wh = W.astype(_bf16)
    Wl = (W - Wh.astype(_f32)).astype(_bf16)
    return Wh, Wl


def _dot3(x, Wh, Wl):
    xh = x.astype(_bf16)
    xl = (x - xh.astype(_f32)).astype(_bf16)
    return (jnp.dot(xh, Wh, preferred_element_type=_f32)
            + jnp.dot(xh, Wl, preferred_element_type=_f32)
            + jnp.dot(xl, Wh, preferred_element_type=_f32))


def _b2x(b):
    return jnp.concatenate([b, b]).reshape(1, 2 * U)


def _mlp(terms, b1, W2, b2, n):
    n_t = len(terms)

    def body(*refs):
        t_refs = refs[:n_t]
        w1_refs = refs[n_t:2 * n_t]
        b1_ref, w2_ref, b2_ref = refs[2 * n_t:2 * n_t + 3]
        o_ref = refs[2 * n_t + 3]
        acc = jnp.broadcast_to(b1_ref[...], (BN // 2, 2 * U)).astype(_f32)
        for t, (arr, kind, _) in enumerate(terms):
            if kind == 'packed':
                x = t_refs[t][...]
            elif kind[0] == 'pairp':
                x = t_refs[t][0] + t_refs[t][1]
            else:
                x = t_refs[t][0]
            acc = acc + jnp.dot(x, w1_refs[t][...], preferred_element_type=_f32)
        h = jnp.tanh(acc)
        o_ref[...] = jnp.dot(h, w2_ref[...],
                             preferred_element_type=_f32) + b2_ref[...]

    in_specs = []
    args = []
    for (arr, kind, _) in terms:
        if kind == 'packed':
            in_specs.append(pl.BlockSpec((BN // 2, 128), lambda i: (i, 0)))
        elif kind[0] == 'pairp':
            in_specs.append(pl.BlockSpec((NC, BN // 2, 128),
                                         lambda i: (0, i, 0)))
        else:
            _, q, qs = kind
            in_specs.append(pl.BlockSpec((1, BN // 2, 128),
                                         lambda i, q=q: (q, i, 0)))
        args.append(arr)
    for (_, _, w1s) in terms:
        in_specs.append(pl.BlockSpec((128, 128), lambda i: (0, 0)))
        args.append(_bd(w1s))
    in_specs += [pl.BlockSpec((1, 128), lambda i: (0, 0)),
                 pl.BlockSpec((128, 128), lambda i: (0, 0)),
                 pl.BlockSpec((1, 128), lambda i: (0, 0))]
    args += [_b2x(b1), _bd(W2), _b2x(b2)]
    return pl.pallas_call(
        body, grid=(pl.cdiv(n, BN),),
        in_specs=in_specs,
        out_specs=pl.BlockSpec((BN // 2, 128), lambda i: (i, 0)),
        out_shape=jax.ShapeDtypeStruct((n * U // 128, 128), _f32))(*args)


def _pair_sum(parts, n):
    def body(p_ref, o_ref):
        o_ref[...] = p_ref[0] + p_ref[1]

    return pl.pallas_call(
        body, grid=(pl.cdiv(n, BN),),
        in_specs=[pl.BlockSpec((NC, BN // 2, 128), lambda i: (0, i, 0))],
        out_specs=pl.BlockSpec((BN // 2, 128), lambda i: (i, 0)),
        out_shape=jax.ShapeDtypeStruct((n * U // 128, 128), _f32))(parts)


def _pack2(x):
    n = x.shape[0]
    return lax.optimization_barrier(x.reshape(n * U // 128, 128))


def _pku(x, n):
    return x.reshape(-1, U)[:n]


def kernel(h1, h2, h3, h4, Wup2_1, bup2_1, Wup2_2, bup2_2, Wup3_1, bup3_1, Wup3_2, bup3_2, Wup4_1, bup4_1, Wup4_2, bup4_2, Wdn1_1, bdn1_1, Wdn1_2, bdn1_2, Wdn2_1, bdn2_1, Wdn2_2, bdn2_2, Wdn3_1, bdn3_1, Wdn3_2, bdn3_2, Wr_1, br_1, Wr_2, br_2, up2_0, up2_1, up3_0, up3_1, up4_0, up4_1, dn3_0, dn3_1, dn2_0, dn2_1, dn1_0, dn1_1, ring_in, ring_out):
    n1, n2, n3, n4 = h1.shape[0], h2.shape[0], h3.shape[0], h4.shape[0]
    NR = 10000

    u20, u21 = _edge_tiles(up2_0, 32, n2, 64), _edge_tiles(up2_1, 32, n2, 64)
    u30, u31 = _edge_tiles(up3_0, 8, n3), _edge_tiles(up3_1, 8, n3)
    u40, u41 = _edge_tiles(up4_0, 4, n4), _edge_tiles(up4_1, 4, n4)
    d30, d31 = _edge_tiles(dn3_0, 4, n3), _edge_tiles(dn3_1, 4, n3)
    d20, d21 = _edge_tiles(dn2_0, 16, n2, 64), _edge_tiles(dn2_1, 16, n2, 64)
    d10, d11 = _edge_tiles(dn1_0, 32, n1, 64), _edge_tiles(dn1_1, 32, n1, 64)
    rin, rout = _edge_tiles(ring_in, 8, NR), _edge_tiles(ring_out, 16, n1, 64)

    h1p, h2p = _pack2(h1), _pack2(h2)
    h3p, h4p = _pack2(h3), _pack2(h4)

    s2 = _seg_row_split(
        _pku(h1p, n1), [[u20], [u21]], n2, 2,
        chunk_of=lambda c, p: c, out_of=lambda c, p: p, n_out=2)
    s2 = s2.reshape(2, -1, 128)
    h2u = _mlp([(h2p, 'packed', Wup2_1[0:64]),
                (s2, ('slotp', 0, 2), Wup2_1[64:128]),
                (s2, ('slotp', 1, 2), Wup2_1[128:192])],
               bup2_1, Wup2_2, bup2_2, n2)

    s3 = _seg_etype_split(_pku(h2u, n2), [u30, u31], n3).reshape(2, -1, 128)
    h3u = _mlp([(h3p, 'packed', Wup3_1[0:64]),
                (s3, ('slotp', 0, 2), Wup3_1[64:128]),
                (s3, ('slotp', 1, 2), Wup3_1[128:192])],
               bup3_1, Wup3_2, bup3_2, n3)

    s4 = _seg_etype_split(_pku(h3u, n3), [u40, u41], n4).reshape(2, -1, 128)
    h4f = _mlp([(h4p, 'packed', Wup4_1[0:64]),
                (s4, ('slotp', 0, 2), Wup4_1[64:128]),
                (s4, ('slotp', 1, 2), Wup4_1[128:192])],
               bup4_1, Wup4_2, bup4_2, n4)

    d3 = _seg_edge_split(_pku(h4f, n4), [d30, d31], n3).reshape(NC, -1, 128)
    h3f = _mlp([(h3u, 'packed', Wdn3_1[0:64]),
                (d3, ('pairp',), Wdn3_1[64:128])],
               bdn3_1, Wdn3_2, bdn3_2, n3)

    d2 = _seg_row_split(
        _pku(h3f, n3), [[d20, d21]], n2, 2,
        chunk_of=lambda c, p: c, out_of=lambda c, p: 0, n_out=1)
    d2 = d2.reshape(1, -1, 128)
    h2f = _mlp([(h2u, 'packed', Wdn2_1[0:64]),
                (d2, ('slotp', 0, 1), Wdn2_1[64:128])],
               bdn2_1, Wdn2_2, bdn2_2, n2)

    d1 = _seg_row_split(
        _pku(h2f, n2), [[d10, d11], [d10, d11]], n1, 4,
        chunk_of=lambda c, p: 2 * c + p, out_of=lambda c, p: 0, n_out=1)
    d1 = d1.reshape(1, -1, 128)
    h1r = _mlp([(h1p, 'packed', Wdn1_1[0:64]),
                (d1, ('slotp', 0, 1), Wdn1_1[64:128])],
               bdn1_1, Wdn1_2, bdn1_2, n1)

    hrp = _seg_edge_split(_pku(h1r, n1), [rin], NR).reshape(NC, -1, 128)
    hr = _pair_sum(hrp, NR)
    r1 = _seg_row_split(
        _pku(hr, NR), [[rout], [rout]], n1, 4,
        chunk_of=lambda c, p: 2 * c + p, out_of=lambda c, p: 0, n_out=1)
    r1 = r1.reshape(1, -1, 128)
    h1f = _mlp([(h1r, 'packed', Wr_1[0:64]),
                (r1, ('slotp', 0, 1), Wr_1[64:128])],
               br_1, Wr_2, br_2, n1)

    return (_pku(h1f, n1), _pku(h2f, n2), _pku(h3f, n3), _pku(h4f, n4),
            _pku(hr, NR))


# speedup vs baseline: 4.5352x; 1.0175x over previous
import jax
import jax.numpy as jnp
from jax import lax
from jax.experimental import pallas as pl
from jax.experimental.pallas import tpu as pltpu
from jax.experimental.pallas import tpu_sc as plsc

U = 64
NC, NS = 2, 16
GB = 128
KPIPE = 6
BN = 8192
ZR = 16
NDUMP = 64

_MESH = plsc.VectorSubcoreMesh(
    core_axis_name="c", subcore_axis_name="s", num_cores=NC, num_subcores=NS)
_SC_PARAMS = pltpu.CompilerParams(use_tc_tiling_on_sc=False)

_f32 = jnp.float32


def _acc_geom(n):
    rpw = -(-n // NS)
    rpw = -(-rpw // ZR) * ZR
    return rpw, NS * rpw


def _chunk_size(n, n_chunks):
    return -(-(-(-n // n_chunks)) // (NS * ZR)) * (NS * ZR)


def _edge_tiles(e, nb, n_dst, gb=GB):
    E = e.shape[1]
    tile = nb * gb
    Ep = -(-E // tile) * tile
    src, dst = e[0], e[1]
    if Ep != E:
        src = jnp.concatenate([src, jnp.zeros((Ep - E,), e.dtype)])
        dst = jnp.concatenate([dst, jnp.full((Ep - E,), n_dst, e.dtype)])
    return src.reshape(-1, nb, gb), dst.reshape(-1, nb, gb)


def _fill_zero(zbuf):
    zr, cols = zbuf.shape
    z = jnp.zeros((16,), _f32)

    @pl.loop(0, zr)
    def _(r):
        @pl.loop(0, cols, step=16)
        def _(cc):
            zbuf[r, pl.ds(cc, 16)] = z


def _zero_rows(acc, zbuf, lo, nrows, sem):
    nz = nrows // ZR

    @pl.loop(0, nz)
    def _(i):
        pltpu.async_copy(zbuf, acc.at[pl.ds(lo + i * ZR, ZR)], sem)

    @pl.loop(0, nz)
    def _(i):
        pltpu.make_async_copy(zbuf, acc.at[pl.ds(lo, ZR)], sem).wait()


def _stream_tiles(table_ref, src_r, dst_r, acc, sbuf, dbuf, rowbufs, gsems,
                  ssems, start, step, clamp=None):
    t_tot, nb, gb = src_r.shape

    D = KPIPE // 2

    @pl.loop(start, t_tot, step=step)
    def _(t):
        pltpu.sync_copy(src_r.at[t], sbuf)
        pltpu.sync_copy(dst_r.at[t], dbuf)
        if clamp is not None:
            lo, cs = clamp

            @pl.loop(0, nb)
            def _(jv):
                for m in range(gb // 16):
                    v = dbuf[jv, pl.ds(16 * m, 16)]
                    inr = (v >= lo) & (v < lo + cs)
                    dump = cs + (v & (NDUMP - 1))
                    dbuf[jv, pl.ds(16 * m, 16)] = jnp.where(inr, v - lo, dump)
        for j in range(min(D, nb)):
            pltpu.async_copy(table_ref.at[sbuf.at[j]], rowbufs[j % KPIPE],
                             gsems[j % KPIPE])
        for j in range(nb):
            k = j % KPIPE
            if j + D < nb:
                kd = (j + D) % KPIPE
                jprev = j + D - KPIPE
                if jprev >= 0:
                    pltpu.make_async_copy(rowbufs[kd], acc.at[dbuf.at[jprev]],
                                          ssems[kd]).wait()
                pltpu.async_copy(table_ref.at[sbuf.at[j + D]], rowbufs[kd],
                                 gsems[kd])
            pltpu.make_async_copy(table_ref.at[sbuf.at[j]], rowbufs[k],
                                  gsems[k]).wait()
            pltpu.async_copy(rowbufs[k], acc.at[dbuf.at[j]], ssems[k],
                             add=True)
        for j in range(max(0, nb - KPIPE), nb):
            k = j % KPIPE
            pltpu.make_async_copy(rowbufs[k], acc.at[dbuf.at[j]],
                                  ssems[k]).wait()


def _sc_scratch(acc_rows, nb, gb):
    return [pltpu.VMEM_SHARED((acc_rows, U), _f32),
            pltpu.VMEM((nb, gb), jnp.int32),
            pltpu.VMEM((nb, gb), jnp.int32)] \
        + [pltpu.VMEM((gb, U), _f32) for _ in range(KPIPE)] \
        + [pltpu.VMEM((ZR, U), _f32)] \
        + [pltpu.SemaphoreType.DMA] * (2 * KPIPE + 1)


def _unpack_refs(refs, n_stream_args):
    stream_refs = refs[:n_stream_args]
    rest = refs[n_stream_args:]
    acc, sbuf, dbuf = rest[:3]
    rowbufs = rest[3:3 + KPIPE]
    zbuf = rest[3 + KPIPE]
    gsems = rest[4 + KPIPE:4 + 2 * KPIPE]
    ssems = rest[4 + 2 * KPIPE:4 + 3 * KPIPE]
    zsem = rest[4 + 3 * KPIPE]
    return stream_refs, acc, sbuf, dbuf, rowbufs, zbuf, gsems, ssems, zsem


def _seg_edge_split(table, streams, n):
    rpw, npad = _acc_geom(n)
    n_str = len(streams)
    nb = streams[0][0].shape[1]

    def body(*refs):
        (srefs, acc, sbuf, dbuf, rowbufs, zbuf, gsems, ssems,
         zsem) = _unpack_refs(refs[1:1 + 2 * n_str] + refs[2 + 2 * n_str:],
                              2 * n_str)
        table_ref = refs[0]
        out = refs[1 + 2 * n_str]
        c = lax.axis_index("c")
        s = lax.axis_index("s")
        gw = c * NS + s
        _fill_zero(zbuf)
        _zero_rows(acc, zbuf, s * rpw, rpw, zsem)
        plsc.subcore_barrier()
        for k in range(n_str):
            _stream_tiles(table_ref, srefs[2 * k], srefs[2 * k + 1],
                          acc, sbuf, dbuf, rowbufs, gsems, ssems, gw, NC * NS)
        plsc.subcore_barrier()
        pltpu.sync_copy(acc.at[pl.ds(s * rpw, rpw)],
                        out.at[c, pl.ds(s * rpw, rpw)])

    out_type = jax.ShapeDtypeStruct((NC, npad, U), _f32)
    fn = pl.kernel(body, out_type=out_type, mesh=_MESH,
                   scratch_types=_sc_scratch(npad + ZR, nb,
                                             streams[0][0].shape[2]),
                   compiler_params=_SC_PARAMS)
    args = [table]
    for (sr, dr) in streams:
        args += [sr, dr]
    return fn(*args)


def _seg_etype_split(table, streams, n):
    rpw, npad = _acc_geom(n)
    nb = streams[0][0].shape[1]

    def body(*refs):
        (srefs, acc, sbuf, dbuf, rowbufs, zbuf, gsems, ssems,
         zsem) = _unpack_refs(refs[1:5] + refs[6:], 4)
        table_ref = refs[0]
        out = refs[5]
        c = lax.axis_index("c")
        s = lax.axis_index("s")
        _fill_zero(zbuf)
        _zero_rows(acc, zbuf, s * rpw, rpw, zsem)
        plsc.subcore_barrier()
        for k in range(2):
            @pl.when(c == k)
            def _(k=k):
                _stream_tiles(table_ref, srefs[2 * k], srefs[2 * k + 1],
                              acc, sbuf, dbuf, rowbufs, gsems, ssems, s, NS)
        plsc.subcore_barrier()
        pltpu.sync_copy(acc.at[pl.ds(s * rpw, rpw)],
                        out.at[c, pl.ds(s * rpw, rpw)])

    out_type = jax.ShapeDtypeStruct((2, npad, U), _f32)
    fn = pl.kernel(body, out_type=out_type, mesh=_MESH,
                   scratch_types=_sc_scratch(npad + ZR, nb,
                                             streams[0][0].shape[2]),
                   compiler_params=_SC_PARAMS)
    return fn(table, streams[0][0], streams[0][1], streams[1][0], streams[1][1])


def _seg_row_split(table, passes, n, n_chunks, chunk_of, out_of, n_out):
    CS = _chunk_size(n, n_chunks)
    rpw_c = CS // NS
    n_pass = len(passes)
    flat = [st for ps in passes for st in ps]
    nb = flat[0][0].shape[1]

    def body(*refs):
        (srefs, acc, sbuf, dbuf, rowbufs, zbuf, gsems, ssems,
         zsem) = _unpack_refs(refs[1:1 + 2 * len(flat)]
                              + refs[2 + 2 * len(flat):], 2 * len(flat))
        table_ref = refs[0]
        out = refs[1 + 2 * len(flat)]
        c = lax.axis_index("c")
        s = lax.axis_index("s")
        _fill_zero(zbuf)
        off = 0
        for p in range(n_pass):
            _zero_rows(acc, zbuf, s * rpw_c, rpw_c, zsem)
            plsc.subcore_barrier()
            q = chunk_of(c, p)
            oidx = out_of(c, p)
            lo = q * CS
            for k in range(len(passes[p])):
                _stream_tiles(table_ref, srefs[2 * (off + k)],
                              srefs[2 * (off + k) + 1],
                              acc, sbuf, dbuf, rowbufs, gsems, ssems,
                              s, NS, clamp=(lo, CS))
            off += len(passes[p])
            plsc.subcore_barrier()
            pltpu.sync_copy(acc.at[pl.ds(s * rpw_c, rpw_c)],
                            out.at[oidx, pl.ds(lo + s * rpw_c, rpw_c)])

    out_type = jax.ShapeDtypeStruct((n_out, n_chunks * CS, U), _f32)
    fn = pl.kernel(body, out_type=out_type, mesh=_MESH,
                   scratch_types=_sc_scratch(CS + NDUMP, nb,
                                             flat[0][0].shape[2]),
                   compiler_params=_SC_PARAMS)
    args = [table]
    for (sr, dr) in flat:
        args += [sr, dr]
    return fn(*args)


def _bd(W):
    k = W.shape[0]
    z = jnp.zeros((k, U), _f32)
    return jnp.concatenate(
        [jnp.concatenate([W, z], axis=1), jnp.concatenate([z, W], axis=1)],
        axis=0)


_bf16 = jnp.bfloat16


def _wsplit(W):
    Wh = W.astype(_bf16)
    Wl = (W - Wh.astype(_f32)).astype(_bf16)
    return Wh, Wl


def _dot3(x, Wh, Wl):
    xh = x.astype(_bf16)
    xl = (x - xh.astype(_f32)).astype(_bf16)
    return (jnp.dot(xh, Wh, preferred_element_type=_f32)
            + jnp.dot(xh, Wl, preferred_element_type=_f32)
            + jnp.dot(xl, Wh, preferred_element_type=_f32))


def _b2x(b):
    return jnp.concatenate([b, b]).reshape(1, 2 * U)


def _mlp(terms, b1, W2, b2, n):
    n_t = len(terms)

    def body(*refs):
        t_refs = refs[:n_t]
        w1_refs = refs[n_t:2 * n_t]
        b1_ref, w2_ref, b2_ref = refs[2 * n_t:2 * n_t + 3]
        o_ref = refs[2 * n_t + 3]
        acc = jnp.broadcast_to(b1_ref[...], (BN // 2, 2 * U)).astype(_f32)
        for t, (arr, kind, _) in enumerate(terms):
            if kind == 'packed':
                x = t_refs[t][...]
            elif kind[0] == 'pairp':
                x = t_refs[t][0] + t_refs[t][1]
            else:
                x = t_refs[t][0]
            acc = acc + jnp.dot(x, w1_refs[t][...], preferred_element_type=_f32)
        h = jnp.tanh(acc)
        o_ref[...] = jnp.dot(h, w2_ref[...],
                             preferred_element_type=_f32) + b2_ref[...]

    in_specs = []
    args = []
    for (arr, kind, _) in terms:
        if kind == 'packed':
            in_specs.append(pl.BlockSpec((BN // 2, 128), lambda i: (i, 0)))
        elif kind[0] == 'pairp':
            in_specs.append(pl.BlockSpec((NC, BN // 2, 128),
                                         lambda i: (0, i, 0)))
        else:
            _, q, qs = kind
            in_specs.append(pl.BlockSpec((1, BN // 2, 128),
                                         lambda i, q=q: (q, i, 0)))
        args.append(arr)
    for (_, _, w1s) in terms:
        in_specs.append(pl.BlockSpec((128, 128), lambda i: (0, 0)))
        args.append(_bd(w1s))
    in_specs += [pl.BlockSpec((1, 128), lambda i: (0, 0)),
                 pl.BlockSpec((128, 128), lambda i: (0, 0)),
                 pl.BlockSpec((1, 128), lambda i: (0, 0))]
    args += [_b2x(b1), _bd(W2), _b2x(b2)]
    return pl.pallas_call(
        body, grid=(pl.cdiv(n, BN),),
        in_specs=in_specs,
        out_specs=pl.BlockSpec((BN // 2, 128), lambda i: (i, 0)),
        out_shape=jax.ShapeDtypeStruct((n * U // 128, 128), _f32))(*args)


def _pair_sum(parts, n):
    def body(p_ref, o_ref):
        o_ref[...] = p_ref[0] + p_ref[1]

    return pl.pallas_call(
        body, grid=(pl.cdiv(n, BN),),
        in_specs=[pl.BlockSpec((NC, BN // 2, 128), lambda i: (0, i, 0))],
        out_specs=pl.BlockSpec((BN // 2, 128), lambda i: (i, 0)),
        out_shape=jax.ShapeDtypeStruct((n * U // 128, 128), _f32))(parts)


def _pack2(x):
    n = x.shape[0]
    return lax.optimization_barrier(x.reshape(n * U // 128, 128))


def _pku(x, n):
    return x.reshape(-1, U)[:n]


def kernel(h1, h2, h3, h4, Wup2_1, bup2_1, Wup2_2, bup2_2, Wup3_1, bup3_1, Wup3_2, bup3_2, Wup4_1, bup4_1, Wup4_2, bup4_2, Wdn1_1, bdn1_1, Wdn1_2, bdn1_2, Wdn2_1, bdn2_1, Wdn2_2, bdn2_2, Wdn3_1, bdn3_1, Wdn3_2, bdn3_2, Wr_1, br_1, Wr_2, br_2, up2_0, up2_1, up3_0, up3_1, up4_0, up4_1, dn3_0, dn3_1, dn2_0, dn2_1, dn1_0, dn1_1, ring_in, ring_out):
    n1, n2, n3, n4 = h1.shape[0], h2.shape[0], h3.shape[0], h4.shape[0]
    NR = 10000

    u20, u21 = _edge_tiles(up2_0, 32, n2, 64), _edge_tiles(up2_1, 32, n2, 64)
    u30, u31 = _edge_tiles(up3_0, 8, n3), _edge_tiles(up3_1, 8, n3)
    u40, u41 = _edge_tiles(up4_0, 4, n4), _edge_tiles(up4_1, 4, n4)
    d30, d31 = _edge_tiles(dn3_0, 4, n3), _edge_tiles(dn3_1, 4, n3)
    d20, d21 = _edge_tiles(dn2_0, 16, n2, 64), _edge_tiles(dn2_1, 16, n2, 64)
    d10, d11 = _edge_tiles(dn1_0, 32, n1, 64), _edge_tiles(dn1_1, 32, n1, 64)
    rin, rout = _edge_tiles(ring_in, 8, NR), _edge_tiles(ring_out, 16, n1, 64)

    h1p, h2p = _pack2(h1), _pack2(h2)
    h3p, h4p = _pack2(h3), _pack2(h4)

    s2 = _seg_row_split(
        _pku(h1p, n1), [[u20], [u21]], n2, 2,
        chunk_of=lambda c, p: c, out_of=lambda c, p: p, n_out=2)
    s2 = s2.reshape(2, -1, 128)
    h2u = _mlp([(h2p, 'packed', Wup2_1[0:64]),
                (s2, ('slotp', 0, 2), Wup2_1[64:128]),
                (s2, ('slotp', 1, 2), Wup2_1[128:192])],
               bup2_1, Wup2_2, bup2_2, n2)

    s3 = _seg_etype_split(_pku(h2u, n2), [u30, u31], n3).reshape(2, -1, 128)
    h3u = _mlp([(h3p, 'packed', Wup3_1[0:64]),
                (s3, ('slotp', 0, 2), Wup3_1[64:128]),
                (s3, ('slotp', 1, 2), Wup3_1[128:192])],
               bup3_1, Wup3_2, bup3_2, n3)

    s4 = _seg_etype_split(_pku(h3u, n3), [u40, u41], n4).reshape(2, -1, 128)
    h4f = _mlp([(h4p, 'packed', Wup4_1[0:64]),
                (s4, ('slotp', 0, 2), Wup4_1[64:128]),
                (s4, ('slotp', 1, 2), Wup4_1[128:192])],
               bup4_1, Wup4_2, bup4_2, n4)

    d3 = _seg_edge_split(_pku(h4f, n4), [d30, d31], n3).reshape(NC, -1, 128)
    h3f = _mlp([(h3u, 'packed', Wdn3_1[0:64]),
                (d3, ('pairp',), Wdn3_1[64:128])],
               bdn3_1, Wdn3_2, bdn3_2, n3)

    d2 = _seg_row_split(
        _pku(h3f, n3), [[d20, d21]], n2, 2,
        chunk_of=lambda c, p: c, out_of=lambda c, p: 0, n_out=1)
    d2 = d2.reshape(1, -1, 128)
    h2f = _mlp([(h2u, 'packed', Wdn2_1[0:64]),
                (d2, ('slotp', 0, 1), Wdn2_1[64:128])],
               bdn2_1, Wdn2_2, bdn2_2, n2)

    d1 = _seg_row_split(
        _pku(h2f, n2), [[d10, d11], [d10, d11]], n1, 4,
        chunk_of=lambda c, p: 2 * c + p, out_of=lambda c, p: 0, n_out=1)
    d1 = d1.reshape(1, -1, 128)
    h1r = _mlp([(h1p, 'packed', Wdn1_1[0:64]),
                (d1, ('slotp', 0, 1), Wdn1_1[64:128])],
               bdn1_1, Wdn1_2, bdn1_2, n1)

    hrp = _seg_edge_split(_pku(h1r, n1), [rin], NR).reshape(NC, -1, 128)
    hr = _pair_sum(hrp, NR)
    r1 = _seg_row_split(
        _pku(hr, NR), [[rout], [rout]], n1, 4,
        chunk_of=lambda c, p: 2 * c + p, out_of=lambda c, p: 0, n_out=1)
    r1 = r1.reshape(1, -1, 128)
    h1f = _mlp([(h1r, 'packed', Wr_1[0:64]),
                (r1, ('slotp', 0, 1), Wr_1[64:128])],
               br_1, Wr_2, br_2, n1)

    return (_pku(h1f, n1), _pku(h2f, n2), _pku(h3f, n3), _pku(h4f, n4),
            _pku(hr, NR))


# speedup vs baseline: 4.5366x; 1.0003x over previous
import jax
import jax.numpy as jnp
from jax import lax
from jax.experimental import pallas as pl
from jax.experimental.pallas import tpu as pltpu
from jax.experimental.pallas import tpu_sc as plsc

U = 64
NC, NS = 2, 16
GB = 128
KPIPE = 6
BN = 8192
ZR = 16
NDUMP = 64

_MESH = plsc.VectorSubcoreMesh(
    core_axis_name="c", subcore_axis_name="s", num_cores=NC, num_subcores=NS)
_SC_PARAMS = pltpu.CompilerParams(use_tc_tiling_on_sc=False)

_f32 = jnp.float32


def _acc_geom(n):
    rpw = -(-n // NS)
    rpw = -(-rpw // ZR) * ZR
    return rpw, NS * rpw


def _chunk_size(n, n_chunks):
    return -(-(-(-n // n_chunks)) // (NS * ZR)) * (NS * ZR)


def _edge_tiles(e, nb, n_dst, gb=GB):
    E = e.shape[1]
    tile = nb * gb
    Ep = -(-E // tile) * tile
    src, dst = e[0], e[1]
    if Ep != E:
        src = jnp.concatenate([src, jnp.zeros((Ep - E,), e.dtype)])
        dst = jnp.concatenate([dst, jnp.full((Ep - E,), n_dst, e.dtype)])
    return src.reshape(-1, nb, gb), dst.reshape(-1, nb, gb)


def _fill_zero(zbuf):
    zr, cols = zbuf.shape
    z = jnp.zeros((16,), _f32)

    @pl.loop(0, zr)
    def _(r):
        @pl.loop(0, cols, step=16)
        def _(cc):
            zbuf[r, pl.ds(cc, 16)] = z


def _zero_rows(acc, zbuf, lo, nrows, sem):
    nz = nrows // ZR

    @pl.loop(0, nz)
    def _(i):
        pltpu.async_copy(zbuf, acc.at[pl.ds(lo + i * ZR, ZR)], sem)

    @pl.loop(0, nz)
    def _(i):
        pltpu.make_async_copy(zbuf, acc.at[pl.ds(lo, ZR)], sem).wait()


def _stream_tiles(table_ref, src_r, dst_r, acc, sbuf, dbuf, rowbufs, gsems,
                  ssems, start, step, clamp=None):
    t_tot, nb, gb = src_r.shape

    D = KPIPE // 2

    @pl.loop(start, t_tot, step=step)
    def _(t):
        pltpu.sync_copy(src_r.at[t], sbuf)
        pltpu.sync_copy(dst_r.at[t], dbuf)
        if clamp is not None:
            lo, cs = clamp

            @pl.loop(0, nb)
            def _(jv):
                for m in range(gb // 16):
                    v = dbuf[jv, pl.ds(16 * m, 16)]
                    inr = (v >= lo) & (v < lo + cs)
                    dump = cs + (v & (NDUMP - 1))
                    dbuf[jv, pl.ds(16 * m, 16)] = jnp.where(inr, v - lo, dump)
        for j in range(min(D, nb)):
            pltpu.async_copy(table_ref.at[sbuf.at[j]], rowbufs[j % KPIPE],
                             gsems[j % KPIPE])
        for j in range(nb):
            k = j % KPIPE
            if j + D < nb:
                kd = (j + D) % KPIPE
                jprev = j + D - KPIPE
                if jprev >= 0:
                    pltpu.make_async_copy(rowbufs[kd], acc.at[dbuf.at[jprev]],
                                          ssems[kd]).wait()
                pltpu.async_copy(table_ref.at[sbuf.at[j + D]], rowbufs[kd],
                                 gsems[kd])
            pltpu.make_async_copy(table_ref.at[sbuf.at[j]], rowbufs[k],
                                  gsems[k]).wait()
            pltpu.async_copy(rowbufs[k], acc.at[dbuf.at[j]], ssems[k],
                             add=True)
        for j in range(max(0, nb - KPIPE), nb):
            k = j % KPIPE
            pltpu.make_async_copy(rowbufs[k], acc.at[dbuf.at[j]],
                                  ssems[k]).wait()


def _sc_scratch(acc_rows, nb, gb):
    return [pltpu.VMEM_SHARED((acc_rows, U), _f32),
            pltpu.VMEM((nb, gb), jnp.int32),
            pltpu.VMEM((nb, gb), jnp.int32)] \
        + [pltpu.VMEM((gb, U), _f32) for _ in range(KPIPE)] \
        + [pltpu.VMEM((ZR, U), _f32)] \
        + [pltpu.SemaphoreType.DMA] * (2 * KPIPE + 1)


def _unpack_refs(refs, n_stream_args):
    stream_refs = refs[:n_stream_args]
    rest = refs[n_stream_args:]
    acc, sbuf, dbuf = rest[:3]
    rowbufs = rest[3:3 + KPIPE]
    zbuf = rest[3 + KPIPE]
    gsems = rest[4 + KPIPE:4 + 2 * KPIPE]
    ssems = rest[4 + 2 * KPIPE:4 + 3 * KPIPE]
    zsem = rest[4 + 3 * KPIPE]
    return stream_refs, acc, sbuf, dbuf, rowbufs, zbuf, gsems, ssems, zsem


def _seg_edge_split(table, streams, n):
    rpw, npad = _acc_geom(n)
    n_str = len(streams)
    nb = streams[0][0].shape[1]

    def body(*refs):
        (srefs, acc, sbuf, dbuf, rowbufs, zbuf, gsems, ssems,
         zsem) = _unpack_refs(refs[1:1 + 2 * n_str] + refs[2 + 2 * n_str:],
                              2 * n_str)
        table_ref = refs[0]
        out = refs[1 + 2 * n_str]
        c = lax.axis_index("c")
        s = lax.axis_index("s")
        gw = c * NS + s
        _fill_zero(zbuf)
        _zero_rows(acc, zbuf, s * rpw, rpw, zsem)
        plsc.subcore_barrier()
        for k in range(n_str):
            _stream_tiles(table_ref, srefs[2 * k], srefs[2 * k + 1],
                          acc, sbuf, dbuf, rowbufs, gsems, ssems, gw, NC * NS)
        plsc.subcore_barrier()
        pltpu.sync_copy(acc.at[pl.ds(s * rpw, rpw)],
                        out.at[c, pl.ds(s * rpw, rpw)])

    out_type = jax.ShapeDtypeStruct((NC, npad, U), _f32)
    fn = pl.kernel(body, out_type=out_type, mesh=_MESH,
                   scratch_types=_sc_scratch(npad + ZR, nb,
                                             streams[0][0].shape[2]),
                   compiler_params=_SC_PARAMS)
    args = [table]
    for (sr, dr) in streams:
        args += [sr, dr]
    return fn(*args)


def _seg_etype_split(table, streams, n):
    rpw, npad = _acc_geom(n)
    nb = streams[0][0].shape[1]

    def body(*refs):
        (srefs, acc, sbuf, dbuf, rowbufs, zbuf, gsems, ssems,
         zsem) = _unpack_refs(refs[1:5] + refs[6:], 4)
        table_ref = refs[0]
        out = refs[5]
        c = lax.axis_index("c")
        s = lax.axis_index("s")
        _fill_zero(zbuf)
        _zero_rows(acc, zbuf, s * rpw, rpw, zsem)
        plsc.subcore_barrier()
        for k in range(2):
            @pl.when(c == k)
            def _(k=k):
                _stream_tiles(table_ref, srefs[2 * k], srefs[2 * k + 1],
                              acc, sbuf, dbuf, rowbufs, gsems, ssems, s, NS)
        plsc.subcore_barrier()
        pltpu.sync_copy(acc.at[pl.ds(s * rpw, rpw)],
                        out.at[c, pl.ds(s * rpw, rpw)])

    out_type = jax.ShapeDtypeStruct((2, npad, U), _f32)
    fn = pl.kernel(body, out_type=out_type, mesh=_MESH,
                   scratch_types=_sc_scratch(npad + ZR, nb,
                                             streams[0][0].shape[2]),
                   compiler_params=_SC_PARAMS)
    return fn(table, streams[0][0], streams[0][1], streams[1][0], streams[1][1])


def _seg_row_split(table, passes, n, n_chunks, chunk_of, out_of, n_out):
    CS = _chunk_size(n, n_chunks)
    rpw_c = CS // NS
    n_pass = len(passes)
    flat = [st for ps in passes for st in ps]
    nb = flat[0][0].shape[1]

    def body(*refs):
        (srefs, acc, sbuf, dbuf, rowbufs, zbuf, gsems, ssems,
         zsem) = _unpack_refs(refs[1:1 + 2 * len(flat)]
                              + refs[2 + 2 * len(flat):], 2 * len(flat))
        table_ref = refs[0]
        out = refs[1 + 2 * len(flat)]
        c = lax.axis_index("c")
        s = lax.axis_index("s")
        _fill_zero(zbuf)
        off = 0
        for p in range(n_pass):
            _zero_rows(acc, zbuf, s * rpw_c, rpw_c, zsem)
            plsc.subcore_barrier()
            q = chunk_of(c, p)
            oidx = out_of(c, p)
            lo = q * CS
            for k in range(len(passes[p])):
                _stream_tiles(table_ref, srefs[2 * (off + k)],
                              srefs[2 * (off + k) + 1],
                              acc, sbuf, dbuf, rowbufs, gsems, ssems,
                              s, NS, clamp=(lo, CS))
            off += len(passes[p])
            plsc.subcore_barrier()
            pltpu.sync_copy(acc.at[pl.ds(s * rpw_c, rpw_c)],
                            out.at[oidx, pl.ds(lo + s * rpw_c, rpw_c)])

    out_type = jax.ShapeDtypeStruct((n_out, n_chunks * CS, U), _f32)
    fn = pl.kernel(body, out_type=out_type, mesh=_MESH,
                   scratch_types=_sc_scratch(CS + NDUMP, nb,
                                             flat[0][0].shape[2]),
                   compiler_params=_SC_PARAMS)
    args = [table]
    for (sr, dr) in flat:
        args += [sr, dr]
    return fn(*args)


def _bd(W):
    k = W.shape[0]
    z = jnp.zeros((k, U), _f32)
    return jnp.concatenate(
        [jnp.concatenate([W, z], axis=1), jnp.concatenate([z, W], axis=1)],
        axis=0)


def _b2x(b):
    return jnp.concatenate([b, b]).reshape(1, 2 * U)


def _mlp(terms, b1, W2, b2, n):
    n_t = len(terms)

    def body(*refs):
        t_refs = refs[:n_t]
        w1_refs = refs[n_t:2 * n_t]
        b1_ref, w2_ref, b2_ref = refs[2 * n_t:2 * n_t + 3]
        o_ref = refs[2 * n_t + 3]
        acc = jnp.broadcast_to(b1_ref[...], (BN // 2, 2 * U)).astype(_f32)
        for t, (arr, kind, _) in enumerate(terms):
            if kind == 'packed':
                x = t_refs[t][...]
            elif kind[0] == 'pairp':
                x = t_refs[t][0] + t_refs[t][1]
            else:
                x = t_refs[t][0]
            acc = acc + jnp.dot(x, w1_refs[t][...], preferred_element_type=_f32)
        h = jnp.tanh(acc)
        o_ref[...] = jnp.dot(h, w2_ref[...],
                             preferred_element_type=_f32) + b2_ref[...]

    in_specs = []
    args = []
    for (arr, kind, _) in terms:
        if kind == 'packed':
            in_specs.append(pl.BlockSpec((BN // 2, 128), lambda i: (i, 0)))
        elif kind[0] == 'pairp':
            in_specs.append(pl.BlockSpec((NC, BN // 2, 128),
                                         lambda i: (0, i, 0)))
        else:
            _, q, qs = kind
            in_specs.append(pl.BlockSpec((1, BN // 2, 128),
                                         lambda i, q=q: (q, i, 0)))
        args.append(arr)
    for (_, _, w1s) in terms:
        in_specs.append(pl.BlockSpec((128, 128), lambda i: (0, 0)))
        args.append(_bd(w1s))
    in_specs += [pl.BlockSpec((1, 128), lambda i: (0, 0)),
                 pl.BlockSpec((128, 128), lambda i: (0, 0)),
                 pl.BlockSpec((1, 128), lambda i: (0, 0))]
    args += [_b2x(b1), _bd(W2), _b2x(b2)]
    return pl.pallas_call(
        body, grid=(pl.cdiv(n, BN),),
        in_specs=in_specs,
        out_specs=pl.BlockSpec((BN // 2, 128), lambda i: (i, 0)),
        out_shape=jax.ShapeDtypeStruct((n * U // 128, 128), _f32))(*args)


def _pair_sum(parts, n):
    def body(p_ref, o_ref):
        o_ref[...] = p_ref[0] + p_ref[1]

    return pl.pallas_call(
        body, grid=(pl.cdiv(n, BN),),
        in_specs=[pl.BlockSpec((NC, BN // 2, 128), lambda i: (0, i, 0))],
        out_specs=pl.BlockSpec((BN // 2, 128), lambda i: (i, 0)),
        out_shape=jax.ShapeDtypeStruct((n * U // 128, 128), _f32))(parts)


def _pack2(x):
    n = x.shape[0]
    return lax.optimization_barrier(x.reshape(n * U // 128, 128))


def _pku(x, n):
    return x.reshape(-1, U)[:n]


def kernel(h1, h2, h3, h4, Wup2_1, bup2_1, Wup2_2, bup2_2, Wup3_1, bup3_1, Wup3_2, bup3_2, Wup4_1, bup4_1, Wup4_2, bup4_2, Wdn1_1, bdn1_1, Wdn1_2, bdn1_2, Wdn2_1, bdn2_1, Wdn2_2, bdn2_2, Wdn3_1, bdn3_1, Wdn3_2, bdn3_2, Wr_1, br_1, Wr_2, br_2, up2_0, up2_1, up3_0, up3_1, up4_0, up4_1, dn3_0, dn3_1, dn2_0, dn2_1, dn1_0, dn1_1, ring_in, ring_out):
    n1, n2, n3, n4 = h1.shape[0], h2.shape[0], h3.shape[0], h4.shape[0]
    NR = 10000

    u20, u21 = _edge_tiles(up2_0, 32, n2, 64), _edge_tiles(up2_1, 32, n2, 64)
    u30, u31 = _edge_tiles(up3_0, 8, n3), _edge_tiles(up3_1, 8, n3)
    u40, u41 = _edge_tiles(up4_0, 4, n4), _edge_tiles(up4_1, 4, n4)
    d30, d31 = _edge_tiles(dn3_0, 4, n3), _edge_tiles(dn3_1, 4, n3)
    d20, d21 = _edge_tiles(dn2_0, 16, n2, 64), _edge_tiles(dn2_1, 16, n2, 64)
    d10, d11 = _edge_tiles(dn1_0, 32, n1, 64), _edge_tiles(dn1_1, 32, n1, 64)
    rin, rout = _edge_tiles(ring_in, 8, NR), _edge_tiles(ring_out, 16, n1, 64)

    h1p, h2p = _pack2(h1), _pack2(h2)
    h3p, h4p = _pack2(h3), _pack2(h4)

    s2 = _seg_row_split(
        _pku(h1p, n1), [[u20], [u21]], n2, 2,
        chunk_of=lambda c, p: c, out_of=lambda c, p: p, n_out=2)
    s2 = s2.reshape(2, -1, 128)
    h2u = _mlp([(h2p, 'packed', Wup2_1[0:64]),
                (s2, ('slotp', 0, 2), Wup2_1[64:128]),
                (s2, ('slotp', 1, 2), Wup2_1[128:192])],
               bup2_1, Wup2_2, bup2_2, n2)

    s3 = _seg_etype_split(_pku(h2u, n2), [u30, u31], n3).reshape(2, -1, 128)
    h3u = _mlp([(h3p, 'packed', Wup3_1[0:64]),
                (s3, ('slotp', 0, 2), Wup3_1[64:128]),
                (s3, ('slotp', 1, 2), Wup3_1[128:192])],
               bup3_1, Wup3_2, bup3_2, n3)

    s4 = _seg_etype_split(_pku(h3u, n3), [u40, u41], n4).reshape(2, -1, 128)
    h4f = _mlp([(h4p, 'packed', Wup4_1[0:64]),
                (s4, ('slotp', 0, 2), Wup4_1[64:128]),
                (s4, ('slotp', 1, 2), Wup4_1[128:192])],
               bup4_1, Wup4_2, bup4_2, n4)

    d3 = _seg_edge_split(_pku(h4f, n4), [d30, d31], n3).reshape(NC, -1, 128)
    h3f = _mlp([(h3u, 'packed', Wdn3_1[0:64]),
                (d3, ('pairp',), Wdn3_1[64:128])],
               bdn3_1, Wdn3_2, bdn3_2, n3)

    d2 = _seg_row_split(
        _pku(h3f, n3), [[d20, d21]], n2, 2,
        chunk_of=lambda c, p: c, out_of=lambda c, p: 0, n_out=1)
    d2 = d2.reshape(1, -1, 128)
    h2f = _mlp([(h2u, 'packed', Wdn2_1[0:64]),
                (d2, ('slotp', 0, 1), Wdn2_1[64:128])],
               bdn2_1, Wdn2_2, bdn2_2, n2)

    d1 = _seg_row_split(
        _pku(h2f, n2), [[d10, d11], [d10, d11]], n1, 4,
        chunk_of=lambda c, p: 2 * c + p, out_of=lambda c, p: 0, n_out=1)
    d1 = d1.reshape(1, -1, 128)
    h1r = _mlp([(h1p, 'packed', Wdn1_1[0:64]),
                (d1, ('slotp', 0, 1), Wdn1_1[64:128])],
               bdn1_1, Wdn1_2, bdn1_2, n1)

    hrp = _seg_edge_split(_pku(h1r, n1), [rin], NR).reshape(NC, -1, 128)
    hr = _pair_sum(hrp, NR)
    r1 = _seg_row_split(
        _pku(hr, NR), [[rout], [rout]], n1, 4,
        chunk_of=lambda c, p: 2 * c + p, out_of=lambda c, p: 0, n_out=1)
    r1 = r1.reshape(1, -1, 128)
    h1f = _mlp([(h1r, 'packed', Wr_1[0:64]),
                (r1, ('slotp', 0, 1), Wr_1[64:128])],
               br_1, Wr_2, br_2, n1)

    return (_pku(h1f, n1), _pku(h2f, n2), _pku(h3f, n3), _pku(h4f, n4),
            _pku(hr, NR))
